# baseline (speedup 1.0000x reference)
import functools

import jax
import jax.numpy as jnp
import numpy as np
from jax import lax
from jax.experimental import pallas as pl
from jax.experimental.pallas import tpu as pltpu
from jax.experimental.pallas import tpu_sc as plsc

S = 2048
D = 1024
H = 16
HD = 64
E = 8
DFF = 1408
SFF = 2816
EPS = 1e-6
ROPE_BASE = 10000.0

RB = 256
BLK = 256
NP_PAD = 4096 + 8 * (BLK - 1)
NP_PAD = ((NP_PAD + BLK - 1) // BLK) * BLK
NB = NP_PAD // BLK

def _rot_weight(wT):
    w4 = wT.reshape(D, H, 2, 32)
    return jnp.concatenate([-w4[:, :, 1], w4[:, :, 0]], axis=2).reshape(D, D)


def _k1_body(x_ref, cos_ref, sin_ref, ln1_ref, wq_ref, wqr_ref, wk_ref,
             wkr_ref, wv_ref, q_ref, k_ref, v_ref):
    x = x_ref[...]
    var = jnp.mean(x * x, axis=-1, keepdims=True)
    xn = ((x * lax.rsqrt(var + EPS)) * ln1_ref[...]).astype(jnp.bfloat16)
    c, s = cos_ref[...], sin_ref[...]
    q = jnp.dot(xn, wq_ref[...], preferred_element_type=jnp.float32)
    qr = jnp.dot(xn, wqr_ref[...], preferred_element_type=jnp.float32)
    q_ref[...] = (q * c + qr * s).astype(jnp.bfloat16)
    k = jnp.dot(xn, wk_ref[...], preferred_element_type=jnp.float32)
    kr = jnp.dot(xn, wkr_ref[...], preferred_element_type=jnp.float32)
    k_ref[...] = (k * c + kr * s).astype(jnp.bfloat16)
    v_ref[...] = jnp.dot(xn, wv_ref[...],
                         preferred_element_type=jnp.float32).astype(jnp.bfloat16)


def _qkv_rope(x, cosE, sinE, ln1_w, wqT, wqTR, wkT, wkTR, wvT):
    row = lambda i: (i, 0)
    full = lambda i: (0, 0)
    return pl.pallas_call(
        _k1_body,
        grid=(S // RB,),
        in_specs=[
            pl.BlockSpec((RB, D), row),
            pl.BlockSpec((RB, D), row),
            pl.BlockSpec((RB, D), row),
            pl.BlockSpec((1, D), full),
            pl.BlockSpec((D, D), full),
            pl.BlockSpec((D, D), full),
            pl.BlockSpec((D, D), full),
            pl.BlockSpec((D, D), full),
            pl.BlockSpec((D, D), full),
        ],
        out_specs=[pl.BlockSpec((RB, D), row)] * 3,
        out_shape=[jax.ShapeDtypeStruct((S, D), jnp.bfloat16)] * 3,
    )(x, cosE, sinE, ln1_w, wqT, wqTR, wkT, wkTR, wvT)


def _attn_body(q_ref, k_ref, v_ref, o_ref):
    q = q_ref[0]
    k = k_ref[0]
    s = lax.dot_general(q, k, (((1,), (1,)), ((), ())),
                        preferred_element_type=jnp.float32) * (1.0 / 8.0)
    qb = pl.program_id(1)
    rows = qb * RB + lax.broadcasted_iota(jnp.int32, (RB, S), 0)
    cols = lax.broadcasted_iota(jnp.int32, (RB, S), 1)
    s = jnp.where(rows >= cols, s, -1e30)
    m = jnp.max(s, axis=-1, keepdims=True)
    p = jnp.exp(s - m)
    p = (p / jnp.sum(p, axis=-1, keepdims=True)).astype(jnp.bfloat16)
    o_ref[0] = lax.dot_general(p, v_ref[0], (((1,), (0,)), ((), ())),
                               preferred_element_type=jnp.float32
                               ).astype(jnp.bfloat16)


def _attention(qh, kh, vh):
    return pl.pallas_call(
        _attn_body,
        grid=(H, S // RB),
        in_specs=[
            pl.BlockSpec((1, RB, HD), lambda h, qb: (h, qb, 0)),
            pl.BlockSpec((1, S, HD), lambda h, qb: (h, 0, 0)),
            pl.BlockSpec((1, S, HD), lambda h, qb: (h, 0, 0)),
        ],
        out_specs=pl.BlockSpec((1, RB, HD), lambda h, qb: (h, qb, 0)),
        out_shape=jax.ShapeDtypeStruct((H, S, HD), jnp.bfloat16),
    )(qh, kh, vh)


def _k3_body(x_ref, ao_ref, ow_ref, ln2_ref, gw_ref, h1_ref, x2_ref, lg_ref):
    proj = jnp.dot(ao_ref[...], ow_ref[...], preferred_element_type=jnp.float32)
    h1 = x_ref[...] + proj
    h1_ref[...] = h1
    var = jnp.mean(h1 * h1, axis=-1, keepdims=True)
    x2 = (h1 * lax.rsqrt(var + EPS)) * ln2_ref[...]
    x2_ref[...] = x2
    lg_ref[...] = jnp.dot(x2, gw_ref[...], preferred_element_type=jnp.float32)


def _oproj_ln2_gate(x, ao, owT, ln2_w, gwT):
    row = lambda i: (i, 0)
    full = lambda i: (0, 0)
    return pl.pallas_call(
        _k3_body,
        grid=(S // RB,),
        in_specs=[
            pl.BlockSpec((RB, D), row),
            pl.BlockSpec((RB, D), row),
            pl.BlockSpec((D, D), full),
            pl.BlockSpec((1, D), full),
            pl.BlockSpec((D, E), full),
        ],
        out_specs=[
            pl.BlockSpec((RB, D), row),
            pl.BlockSpec((RB, D), row),
            pl.BlockSpec((RB, E), row),
        ],
        out_shape=[
            jax.ShapeDtypeStruct((S, D), jnp.float32),
            jax.ShapeDtypeStruct((S, D), jnp.float32),
            jax.ShapeDtypeStruct((S, E), jnp.float32),
        ],
    )(x, ao, owT, ln2_w, gwT)


def _silu(a):
    return a * (1.0 / (1.0 + jnp.exp(-a)))


def _moe_body(nlive_ref, be_ref, xg_ref, eg_ref, eu_ref, ed_ref, yg_ref):
    @pl.when(pl.program_id(0) < nlive_ref[0])
    def _():
        xb = xg_ref[...].astype(jnp.bfloat16)
        a = lax.dot_general(xb, eg_ref[0], (((1,), (1,)), ((), ())),
                            preferred_element_type=jnp.float32)
        u = lax.dot_general(xb, eu_ref[0], (((1,), (1,)), ((), ())),
                            preferred_element_type=jnp.float32)
        s = (_silu(a) * u).astype(jnp.bfloat16)
        yg_ref[...] = lax.dot_general(s, ed_ref[0], (((1,), (1,)), ((), ())),
                                      preferred_element_type=jnp.float32)


def _grouped_ffn(xg, egb, eub, edb, be, nlive):
    grid_spec = pltpu.PrefetchScalarGridSpec(
        num_scalar_prefetch=2,
        grid=(NB,),
        in_specs=[
            pl.BlockSpec((BLK, D), lambda b, nl, be: (b, 0)),
            pl.BlockSpec((1, DFF, D), lambda b, nl, be: (be[b], 0, 0)),
            pl.BlockSpec((1, DFF, D), lambda b, nl, be: (be[b], 0, 0)),
            pl.BlockSpec((1, D, DFF), lambda b, nl, be: (be[b], 0, 0)),
        ],
        out_specs=pl.BlockSpec((BLK, D), lambda b, nl, be: (b, 0)),
    )
    return pl.pallas_call(
        _moe_body,
        grid_spec=grid_spec,
        out_shape=jax.ShapeDtypeStruct((NP_PAD, D), jnp.float32),
    )(nlive, be, xg, egb, eub, edb)


def _shared_body(h1_ref, x2_ref, sg_ref, su_ref, sd_ref, o_ref):
    xb = x2_ref[...].astype(jnp.bfloat16)
    a = jnp.dot(xb, sg_ref[...], preferred_element_type=jnp.float32)
    u = jnp.dot(xb, su_ref[...], preferred_element_type=jnp.float32)
    s = (_silu(a) * u).astype(jnp.bfloat16)
    o_ref[...] = h1_ref[...] + jnp.dot(s, sd_ref[...],
                                       preferred_element_type=jnp.float32)


def _shared_ffn(h1, x2, sgT, suT, sdT):
    row = lambda i: (i, 0)
    full = lambda i: (0, 0)
    return pl.pallas_call(
        _shared_body,
        grid=(S // RB,),
        in_specs=[
            pl.BlockSpec((RB, D), row),
            pl.BlockSpec((RB, D), row),
            pl.BlockSpec((D, SFF), full),
            pl.BlockSpec((D, SFF), full),
            pl.BlockSpec((SFF, D), full),
        ],
        out_specs=pl.BlockSpec((RB, D), row),
        out_shape=jax.ShapeDtypeStruct((S, D), jnp.float32),
    )(h1, x2, sgT, suT, sdT)


def _combine_body(base_ref, g0_ref, g1_ref, w0_ref, w1_ref, o_ref):
    o_ref[...] = (base_ref[...] + w0_ref[...] * g0_ref[...]
                  + w1_ref[...] * g1_ref[...])


def _combine(base, g0, g1, w0, w1):
    row = lambda i: (i, 0)
    return pl.pallas_call(
        _combine_body,
        grid=(S // RB,),
        in_specs=[
            pl.BlockSpec((RB, D), row),
            pl.BlockSpec((RB, D), row),
            pl.BlockSpec((RB, D), row),
            pl.BlockSpec((RB, 1), row),
            pl.BlockSpec((RB, 1), row),
        ],
        out_specs=pl.BlockSpec((RB, D), row),
        out_shape=jax.ShapeDtypeStruct((S, D), jnp.float32),
    )(base, g0, g1, w0, w1)


def _sc_gather_rows(table, idx, chunk=32, nbuf=3):
    info = plsc.get_sparse_core_info()
    nw = info.num_cores * info.num_subcores
    n, tail = idx.shape[0], table.shape[1:]
    per_w = n // nw
    n_ch = per_w // chunk
    assert n_ch * chunk == per_w
    mesh = plsc.VectorSubcoreMesh(core_axis_name="c", subcore_axis_name="s")

    @functools.partial(
        pl.kernel, mesh=mesh,
        out_type=jax.ShapeDtypeStruct((n,) + tail, jnp.float32),
        scratch_types=(
            [pltpu.VMEM((per_w,), jnp.int32)]
            + [pltpu.VMEM((chunk,) + tail, jnp.float32)] * nbuf
            + [pltpu.SemaphoreType.DMA] * (2 * nbuf)
        ),
    )
    def k(table_hbm, idx_hbm, out_hbm, idx_v, *bufs_sems):
        bufs = bufs_sems[:nbuf]
        gsems = bufs_sems[nbuf:2 * nbuf]
        wsems = bufs_sems[2 * nbuf:]
        wid = lax.axis_index("s") * info.num_cores + lax.axis_index("c")
        base = wid * per_w
        pltpu.sync_copy(idx_hbm.at[pl.ds(base, per_w)], idx_v)

        def start_gather(c, b):
            return pltpu.async_copy(
                table_hbm.at[idx_v.at[pl.ds(c * chunk, chunk)]],
                bufs[b], gsems[b])

        gh, wh = {}, {}
        for c in range(min(nbuf, n_ch)):
            gh[c] = start_gather(c, c % nbuf)
        for c in range(n_ch):
            b = c % nbuf
            gh[c].wait()
            wh[c] = pltpu.async_copy(
                bufs[b], out_hbm.at[pl.ds(base + c * chunk, chunk)], wsems[b])
            if c + nbuf < n_ch:
                wh[c].wait()
                gh[c + nbuf] = start_gather(c + nbuf, b)
        for c in range(max(0, n_ch - nbuf), n_ch):
            wh[c].wait()

    return k(table, idx)


def kernel(hidden_states, position_ids, ln1_w, q_w, k_w, v_w, o_w, ln2_w,
           gate_w, eg, eu, ed, sg, su, sd):
    x = hidden_states.reshape(S, D)

    inv_freq = 1.0 / (ROPE_BASE ** (jnp.arange(0, HD, 2, dtype=jnp.float32) / HD))
    freqs = jnp.outer(jnp.arange(S, dtype=jnp.float32), inv_freq)
    emb = jnp.concatenate([freqs, freqs], axis=-1)
    pos = position_ids.reshape(S)
    cosE = jnp.tile(jnp.cos(emb)[pos], (1, H))
    sinE = jnp.tile(jnp.sin(emb)[pos], (1, H))

    wqT = q_w.T.astype(jnp.bfloat16)
    wkT = k_w.T.astype(jnp.bfloat16)
    wvT = v_w.T.astype(jnp.bfloat16)
    q, k, v = _qkv_rope(x, cosE, sinE, ln1_w.reshape(1, D), wqT,
                        _rot_weight(wqT), wkT, _rot_weight(wkT), wvT)

    qh = q.reshape(S, H, HD).transpose(1, 0, 2)
    kh = k.reshape(S, H, HD).transpose(1, 0, 2)
    vh = v.reshape(S, H, HD).transpose(1, 0, 2)
    ao = _attention(qh, kh, vh).transpose(1, 0, 2).reshape(S, D)

    h1, x2, logits = _oproj_ln2_gate(x, ao, o_w.T.astype(jnp.bfloat16),
                                     ln2_w.reshape(1, D), gate_w.T)

    scores = jax.nn.softmax(logits, axis=-1)
    topk_w, topk_idx = jax.lax.top_k(scores, 2)
    e_flat = topk_idx.reshape(-1)
    onehot = (e_flat[:, None] == jnp.arange(E)[None, :]).astype(jnp.int32)
    csum = jnp.cumsum(onehot, axis=0) - onehot
    rank = jnp.take_along_axis(csum, e_flat[:, None], axis=1)[:, 0]
    cnt = onehot.sum(0)
    pc = ((cnt + BLK - 1) // BLK) * BLK
    ps = jnp.concatenate([jnp.zeros(1, jnp.int32),
                          jnp.cumsum(pc)[:-1].astype(jnp.int32)])
    dst = ps[e_flat] + rank
    gather_idx = jnp.zeros(NP_PAD, jnp.int32).at[dst].set(
        jnp.arange(4096, dtype=jnp.int32) // 2)
    bpos = jnp.arange(NB, dtype=jnp.int32) * BLK
    ends = (ps + pc)[None, :]
    be = jnp.minimum(jnp.sum((bpos[:, None] >= ends).astype(jnp.int32),
                             axis=1), E - 1).astype(jnp.int32)
    nlive = jnp.array([0], jnp.int32) + (jnp.sum(pc) + BLK - 1) // BLK

    xg = _sc_gather_rows(x2.reshape(S, 8, 128), gather_idx).reshape(NP_PAD, D)
    yg = _grouped_ffn(xg, eg.astype(jnp.bfloat16), eu.astype(jnp.bfloat16),
                      ed.astype(jnp.bfloat16), be, nlive)
    back_idx = jnp.concatenate([dst[0::2], dst[1::2]])
    gathered = _sc_gather_rows(yg, back_idx)
    g0 = gathered[:S]
    g1 = gathered[S:]

    base = _shared_ffn(h1, x2, sg.T.astype(jnp.bfloat16),
                       su.T.astype(jnp.bfloat16), sd.T.astype(jnp.bfloat16))
    out = _combine(base, g0, g1, topk_w[:, 0:1], topk_w[:, 1:2])
    return out.reshape(1, S, D)


# speedup vs baseline: 1.1468x; 1.1468x over previous
import functools

import jax
import jax.numpy as jnp
import numpy as np
from jax import lax
from jax.experimental import pallas as pl
from jax.experimental.pallas import tpu as pltpu
from jax.experimental.pallas import tpu_sc as plsc

S = 2048
D = 1024
H = 16
HD = 64
E = 8
DFF = 1408
SFF = 2816
EPS = 1e-6
ROPE_BASE = 10000.0

RB = 256
BLK = 256
NP_PAD = 4096 + 8 * (BLK - 1)
NP_PAD = ((NP_PAD + BLK - 1) // BLK) * BLK
NB = NP_PAD // BLK

def _rot_weight(wT):
    w4 = wT.reshape(D, H, 2, 32)
    return jnp.concatenate([-w4[:, :, 1], w4[:, :, 0]], axis=2).reshape(D, D)


def _k1_body(x_ref, cos_ref, sin_ref, ln1_ref, wq_ref, wqr_ref, wk_ref,
             wkr_ref, wv_ref, q_ref, k_ref, v_ref):
    x = x_ref[...]
    var = jnp.mean(x * x, axis=-1, keepdims=True)
    xn = ((x * lax.rsqrt(var + EPS)) * ln1_ref[...]).astype(jnp.bfloat16)
    c, s = cos_ref[...], sin_ref[...]
    q = jnp.dot(xn, wq_ref[...], preferred_element_type=jnp.float32)
    qr = jnp.dot(xn, wqr_ref[...], preferred_element_type=jnp.float32)
    q_ref[...] = (q * c + qr * s).astype(jnp.bfloat16)
    k = jnp.dot(xn, wk_ref[...], preferred_element_type=jnp.float32)
    kr = jnp.dot(xn, wkr_ref[...], preferred_element_type=jnp.float32)
    k_ref[...] = (k * c + kr * s).astype(jnp.bfloat16)
    v_ref[...] = jnp.dot(xn, wv_ref[...],
                         preferred_element_type=jnp.float32).astype(jnp.bfloat16)


def _qkv_rope(x, cosE, sinE, ln1_w, wqT, wqTR, wkT, wkTR, wvT):
    row = lambda i: (i, 0)
    full = lambda i: (0, 0)
    return pl.pallas_call(
        _k1_body,
        grid=(S // RB,),
        in_specs=[
            pl.BlockSpec((RB, D), row),
            pl.BlockSpec((RB, D), row),
            pl.BlockSpec((RB, D), row),
            pl.BlockSpec((1, D), full),
            pl.BlockSpec((D, D), full),
            pl.BlockSpec((D, D), full),
            pl.BlockSpec((D, D), full),
            pl.BlockSpec((D, D), full),
            pl.BlockSpec((D, D), full),
        ],
        out_specs=[pl.BlockSpec((RB, D), row)] * 3,
        out_shape=[jax.ShapeDtypeStruct((S, D), jnp.bfloat16)] * 3,
    )(x, cosE, sinE, ln1_w, wqT, wqTR, wkT, wkTR, wvT)


def _attn_body(q_ref, k_ref, v_ref, o_ref):
    q = q_ref[0]
    k = k_ref[0]
    s = lax.dot_general(q, k, (((1,), (1,)), ((), ())),
                        preferred_element_type=jnp.float32) * (1.0 / 8.0)
    qb = pl.program_id(1)
    rows = qb * RB + lax.broadcasted_iota(jnp.int32, (RB, S), 0)
    cols = lax.broadcasted_iota(jnp.int32, (RB, S), 1)
    s = jnp.where(rows >= cols, s, -1e30)
    m = jnp.max(s, axis=-1, keepdims=True)
    p = jnp.exp(s - m)
    p = (p / jnp.sum(p, axis=-1, keepdims=True)).astype(jnp.bfloat16)
    o_ref[0] = lax.dot_general(p, v_ref[0], (((1,), (0,)), ((), ())),
                               preferred_element_type=jnp.float32
                               ).astype(jnp.bfloat16)


def _attention(qh, kh, vh):
    return pl.pallas_call(
        _attn_body,
        grid=(H, S // RB),
        in_specs=[
            pl.BlockSpec((1, RB, HD), lambda h, qb: (h, qb, 0)),
            pl.BlockSpec((1, S, HD), lambda h, qb: (h, 0, 0)),
            pl.BlockSpec((1, S, HD), lambda h, qb: (h, 0, 0)),
        ],
        out_specs=pl.BlockSpec((1, RB, HD), lambda h, qb: (h, qb, 0)),
        out_shape=jax.ShapeDtypeStruct((H, S, HD), jnp.bfloat16),
    )(qh, kh, vh)


def _k3_body(x_ref, ao_ref, ow_ref, ln2_ref, gw_ref, h1_ref, x2_ref, lg_ref):
    proj = jnp.dot(ao_ref[...], ow_ref[...], preferred_element_type=jnp.float32)
    h1 = x_ref[...] + proj
    h1_ref[...] = h1
    var = jnp.mean(h1 * h1, axis=-1, keepdims=True)
    x2 = (h1 * lax.rsqrt(var + EPS)) * ln2_ref[...]
    x2_ref[...] = x2
    lg_ref[...] = jnp.dot(x2, gw_ref[...], preferred_element_type=jnp.float32)


def _oproj_ln2_gate(x, ao, owT, ln2_w, gwT):
    row = lambda i: (i, 0)
    full = lambda i: (0, 0)
    return pl.pallas_call(
        _k3_body,
        grid=(S // RB,),
        in_specs=[
            pl.BlockSpec((RB, D), row),
            pl.BlockSpec((RB, D), row),
            pl.BlockSpec((D, D), full),
            pl.BlockSpec((1, D), full),
            pl.BlockSpec((D, E), full),
        ],
        out_specs=[
            pl.BlockSpec((RB, D), row),
            pl.BlockSpec((RB, D), row),
            pl.BlockSpec((RB, E), row),
        ],
        out_shape=[
            jax.ShapeDtypeStruct((S, D), jnp.float32),
            jax.ShapeDtypeStruct((S, D), jnp.float32),
            jax.ShapeDtypeStruct((S, E), jnp.float32),
        ],
    )(x, ao, owT, ln2_w, gwT)


def _silu(a):
    return a * (1.0 / (1.0 + jnp.exp(-a)))


def _moe_body(nlive_ref, be_ref, xg_ref, eg_ref, eu_ref, ed_ref, yg_ref):
    @pl.when(pl.program_id(0) < nlive_ref[0])
    def _():
        xb = xg_ref[...].astype(jnp.bfloat16)
        a = lax.dot_general(xb, eg_ref[0], (((1,), (1,)), ((), ())),
                            preferred_element_type=jnp.float32)
        u = lax.dot_general(xb, eu_ref[0], (((1,), (1,)), ((), ())),
                            preferred_element_type=jnp.float32)
        s = (_silu(a) * u).astype(jnp.bfloat16)
        yg_ref[...] = lax.dot_general(s, ed_ref[0], (((1,), (1,)), ((), ())),
                                      preferred_element_type=jnp.float32)


def _grouped_ffn(xg, egb, eub, edb, be, nlive):
    grid_spec = pltpu.PrefetchScalarGridSpec(
        num_scalar_prefetch=2,
        grid=(NB,),
        in_specs=[
            pl.BlockSpec((BLK, D), lambda b, nl, be: (b, 0)),
            pl.BlockSpec((1, DFF, D), lambda b, nl, be: (be[b], 0, 0)),
            pl.BlockSpec((1, DFF, D), lambda b, nl, be: (be[b], 0, 0)),
            pl.BlockSpec((1, D, DFF), lambda b, nl, be: (be[b], 0, 0)),
        ],
        out_specs=pl.BlockSpec((BLK, D), lambda b, nl, be: (b, 0)),
    )
    return pl.pallas_call(
        _moe_body,
        grid_spec=grid_spec,
        out_shape=jax.ShapeDtypeStruct((NP_PAD, D), jnp.float32),
    )(nlive, be, xg, egb, eub, edb)


def _shared_body(h1_ref, x2_ref, sg_ref, su_ref, sd_ref, o_ref):
    xb = x2_ref[...].astype(jnp.bfloat16)
    a = jnp.dot(xb, sg_ref[...], preferred_element_type=jnp.float32)
    u = jnp.dot(xb, su_ref[...], preferred_element_type=jnp.float32)
    s = (_silu(a) * u).astype(jnp.bfloat16)
    o_ref[...] = h1_ref[...] + jnp.dot(s, sd_ref[...],
                                       preferred_element_type=jnp.float32)


def _shared_ffn(h1, x2, sgT, suT, sdT):
    row = lambda i: (i, 0)
    full = lambda i: (0, 0)
    return pl.pallas_call(
        _shared_body,
        grid=(S // RB,),
        in_specs=[
            pl.BlockSpec((RB, D), row),
            pl.BlockSpec((RB, D), row),
            pl.BlockSpec((D, SFF), full),
            pl.BlockSpec((D, SFF), full),
            pl.BlockSpec((SFF, D), full),
        ],
        out_specs=pl.BlockSpec((RB, D), row),
        out_shape=jax.ShapeDtypeStruct((S, D), jnp.float32),
    )(h1, x2, sgT, suT, sdT)


def _combine_body(base_ref, g0_ref, g1_ref, w0_ref, w1_ref, o_ref):
    o_ref[...] = (base_ref[...] + w0_ref[...] * g0_ref[...]
                  + w1_ref[...] * g1_ref[...])


def _combine(base, g0, g1, w0, w1):
    row = lambda i: (i, 0)
    return pl.pallas_call(
        _combine_body,
        grid=(S // RB,),
        in_specs=[
            pl.BlockSpec((RB, D), row),
            pl.BlockSpec((RB, D), row),
            pl.BlockSpec((RB, D), row),
            pl.BlockSpec((RB, 1), row),
            pl.BlockSpec((RB, 1), row),
        ],
        out_specs=pl.BlockSpec((RB, D), row),
        out_shape=jax.ShapeDtypeStruct((S, D), jnp.float32),
    )(base, g0, g1, w0, w1)


def _sc_gather_rows(table, idx, chunk=32, nbuf=3):
    info = plsc.get_sparse_core_info()
    nw = info.num_cores * info.num_subcores
    n, tail = idx.shape[0], table.shape[1:]
    per_w = n // nw
    n_ch = per_w // chunk
    assert n_ch * chunk == per_w
    mesh = plsc.VectorSubcoreMesh(core_axis_name="c", subcore_axis_name="s")

    @functools.partial(
        pl.kernel, mesh=mesh,
        out_type=jax.ShapeDtypeStruct((n,) + tail, jnp.float32),
        scratch_types=(
            [pltpu.VMEM((per_w,), jnp.int32)]
            + [pltpu.VMEM((chunk,) + tail, jnp.float32)] * nbuf
            + [pltpu.SemaphoreType.DMA] * (2 * nbuf)
        ),
    )
    def k(table_hbm, idx_hbm, out_hbm, idx_v, *bufs_sems):
        bufs = bufs_sems[:nbuf]
        gsems = bufs_sems[nbuf:2 * nbuf]
        wsems = bufs_sems[2 * nbuf:]
        wid = lax.axis_index("s") * info.num_cores + lax.axis_index("c")
        base = wid * per_w
        pltpu.sync_copy(idx_hbm.at[pl.ds(base, per_w)], idx_v)

        def start_gather(c, b):
            return pltpu.async_copy(
                table_hbm.at[idx_v.at[pl.ds(c * chunk, chunk)]],
                bufs[b], gsems[b])

        gh, wh = {}, {}
        for c in range(min(nbuf, n_ch)):
            gh[c] = start_gather(c, c % nbuf)
        for c in range(n_ch):
            b = c % nbuf
            gh[c].wait()
            wh[c] = pltpu.async_copy(
                bufs[b], out_hbm.at[pl.ds(base + c * chunk, chunk)], wsems[b])
            if c + nbuf < n_ch:
                wh[c].wait()
                gh[c + nbuf] = start_gather(c + nbuf, b)
        for c in range(max(0, n_ch - nbuf), n_ch):
            wh[c].wait()

    return k(table, idx)


def kernel(hidden_states, position_ids, ln1_w, q_w, k_w, v_w, o_w, ln2_w,
           gate_w, eg, eu, ed, sg, su, sd):
    x = hidden_states.reshape(S, D)

    inv_freq = 1.0 / (ROPE_BASE ** (jnp.arange(0, HD, 2, dtype=jnp.float32) / HD))
    freqs = jnp.outer(jnp.arange(S, dtype=jnp.float32), inv_freq)
    emb = jnp.concatenate([freqs, freqs], axis=-1)
    pos = position_ids.reshape(S)
    cosE = jnp.tile(jnp.cos(emb)[pos], (1, H))
    sinE = jnp.tile(jnp.sin(emb)[pos], (1, H))

    wqT = q_w.T.astype(jnp.bfloat16)
    wkT = k_w.T.astype(jnp.bfloat16)
    wvT = v_w.T.astype(jnp.bfloat16)
    q, k, v = _qkv_rope(x, cosE, sinE, ln1_w.reshape(1, D), wqT,
                        _rot_weight(wqT), wkT, _rot_weight(wkT), wvT)

    qh = q.reshape(S, H, HD).transpose(1, 0, 2)
    kh = k.reshape(S, H, HD).transpose(1, 0, 2)
    vh = v.reshape(S, H, HD).transpose(1, 0, 2)
    ao = _attention(qh, kh, vh).transpose(1, 0, 2).reshape(S, D)

    h1, x2, logits = _oproj_ln2_gate(x, ao, o_w.T.astype(jnp.bfloat16),
                                     ln2_w.reshape(1, D), gate_w.T)

    scores = jax.nn.softmax(logits, axis=-1)
    topk_w, topk_idx = jax.lax.top_k(scores, 2)
    e_flat = topk_idx.reshape(-1)
    onehot = (e_flat[:, None] == jnp.arange(E)[None, :]).astype(jnp.int32)
    csum = jnp.cumsum(onehot, axis=0) - onehot
    rank = jnp.take_along_axis(csum, e_flat[:, None], axis=1)[:, 0]
    cnt = onehot.sum(0)
    pc = ((cnt + BLK - 1) // BLK) * BLK
    ps = jnp.concatenate([jnp.zeros(1, jnp.int32),
                          jnp.cumsum(pc)[:-1].astype(jnp.int32)])
    dst = ps[e_flat] + rank
    gather_idx = (jnp.arange(NP_PAD, dtype=jnp.int32) % S).at[dst].set(
        jnp.arange(4096, dtype=jnp.int32) // 2)
    bpos = jnp.arange(NB, dtype=jnp.int32) * BLK
    ends = (ps + pc)[None, :]
    be = jnp.minimum(jnp.sum((bpos[:, None] >= ends).astype(jnp.int32),
                             axis=1), E - 1).astype(jnp.int32)
    nlive = jnp.array([0], jnp.int32) + (jnp.sum(pc) + BLK - 1) // BLK

    xg = _sc_gather_rows(x2.reshape(S, 8, 128), gather_idx).reshape(NP_PAD, D)
    yg = _grouped_ffn(xg, eg.astype(jnp.bfloat16), eu.astype(jnp.bfloat16),
                      ed.astype(jnp.bfloat16), be, nlive)
    back_idx = jnp.concatenate([dst[0::2], dst[1::2]])
    gathered = _sc_gather_rows(yg, back_idx)
    g0 = gathered[:S]
    g1 = gathered[S:]

    base = _shared_ffn(h1, x2, sg.T.astype(jnp.bfloat16),
                       su.T.astype(jnp.bfloat16), sd.T.astype(jnp.bfloat16))
    out = _combine(base, g0, g1, topk_w[:, 0:1], topk_w[:, 1:2])
    return out.reshape(1, S, D)


# speedup vs baseline: 1.1842x; 1.0326x over previous
import functools

import jax
import jax.numpy as jnp
import numpy as np
from jax import lax
from jax.experimental import pallas as pl
from jax.experimental.pallas import tpu as pltpu
from jax.experimental.pallas import tpu_sc as plsc

S = 2048
D = 1024
H = 16
HD = 64
E = 8
DFF = 1408
SFF = 2816
EPS = 1e-6
ROPE_BASE = 10000.0

RB = 256
BLK = 256
NP_PAD = 4096 + 8 * (BLK - 1)
NP_PAD = ((NP_PAD + BLK - 1) // BLK) * BLK
NB = NP_PAD // BLK

def _rot_weight(wT):
    w4 = wT.reshape(D, H, 2, 32)
    return jnp.concatenate([-w4[:, :, 1], w4[:, :, 0]], axis=2).reshape(D, D)


def _k1_body(x_ref, cos_ref, sin_ref, ln1_ref, wq_ref, wqr_ref, wk_ref,
             wkr_ref, wv_ref, q_ref, k_ref, v_ref):
    x = x_ref[...]
    var = jnp.mean(x * x, axis=-1, keepdims=True)
    xn = ((x * lax.rsqrt(var + EPS)) * ln1_ref[...]).astype(jnp.bfloat16)
    c, s = cos_ref[...], sin_ref[...]
    q = jnp.dot(xn, wq_ref[...], preferred_element_type=jnp.float32)
    qr = jnp.dot(xn, wqr_ref[...], preferred_element_type=jnp.float32)
    q_ref[...] = (q * c + qr * s).astype(jnp.bfloat16)
    k = jnp.dot(xn, wk_ref[...], preferred_element_type=jnp.float32)
    kr = jnp.dot(xn, wkr_ref[...], preferred_element_type=jnp.float32)
    k_ref[...] = (k * c + kr * s).astype(jnp.bfloat16)
    v_ref[...] = jnp.dot(xn, wv_ref[...],
                         preferred_element_type=jnp.float32).astype(jnp.bfloat16)


def _qkv_rope(x, cosE, sinE, ln1_w, wqT, wqTR, wkT, wkTR, wvT):
    row = lambda i: (i, 0)
    full = lambda i: (0, 0)
    return pl.pallas_call(
        _k1_body,
        grid=(S // RB,),
        in_specs=[
            pl.BlockSpec((RB, D), row),
            pl.BlockSpec((RB, D), row),
            pl.BlockSpec((RB, D), row),
            pl.BlockSpec((1, D), full),
            pl.BlockSpec((D, D), full),
            pl.BlockSpec((D, D), full),
            pl.BlockSpec((D, D), full),
            pl.BlockSpec((D, D), full),
            pl.BlockSpec((D, D), full),
        ],
        out_specs=[pl.BlockSpec((RB, D), row)] * 3,
        out_shape=[jax.ShapeDtypeStruct((S, D), jnp.bfloat16)] * 3,
    )(x, cosE, sinE, ln1_w, wqT, wqTR, wkT, wkTR, wvT)


def _attn_body(q_ref, k_ref, v_ref, o_ref):
    q = q_ref[0]
    k = k_ref[0]
    s = lax.dot_general(q, k, (((1,), (1,)), ((), ())),
                        preferred_element_type=jnp.float32) * (1.0 / 8.0)
    qb = pl.program_id(1)
    rows = qb * RB + lax.broadcasted_iota(jnp.int32, (RB, S), 0)
    cols = lax.broadcasted_iota(jnp.int32, (RB, S), 1)
    s = jnp.where(rows >= cols, s, -1e30)
    m = jnp.max(s, axis=-1, keepdims=True)
    p = jnp.exp(s - m)
    p = (p / jnp.sum(p, axis=-1, keepdims=True)).astype(jnp.bfloat16)
    o_ref[0] = lax.dot_general(p, v_ref[0], (((1,), (0,)), ((), ())),
                               preferred_element_type=jnp.float32
                               ).astype(jnp.bfloat16)


def _attention(qh, kh, vh):
    return pl.pallas_call(
        _attn_body,
        grid=(H, S // RB),
        in_specs=[
            pl.BlockSpec((1, RB, HD), lambda h, qb: (h, qb, 0)),
            pl.BlockSpec((1, S, HD), lambda h, qb: (h, 0, 0)),
            pl.BlockSpec((1, S, HD), lambda h, qb: (h, 0, 0)),
        ],
        out_specs=pl.BlockSpec((1, RB, HD), lambda h, qb: (h, qb, 0)),
        out_shape=jax.ShapeDtypeStruct((H, S, HD), jnp.bfloat16),
    )(qh, kh, vh)


def _k3_body(x_ref, ao_ref, ow_ref, ln2_ref, gw_ref, h1_ref, x2_ref, lg_ref):
    proj = jnp.dot(ao_ref[...], ow_ref[...], preferred_element_type=jnp.float32)
    h1 = x_ref[...] + proj
    h1_ref[...] = h1
    var = jnp.mean(h1 * h1, axis=-1, keepdims=True)
    x2 = (h1 * lax.rsqrt(var + EPS)) * ln2_ref[...]
    x2_ref[...] = x2
    lg_ref[...] = jnp.dot(x2, gw_ref[...], preferred_element_type=jnp.float32)


def _oproj_ln2_gate(x, ao, owT, ln2_w, gwT):
    row = lambda i: (i, 0)
    full = lambda i: (0, 0)
    return pl.pallas_call(
        _k3_body,
        grid=(S // RB,),
        in_specs=[
            pl.BlockSpec((RB, D), row),
            pl.BlockSpec((RB, D), row),
            pl.BlockSpec((D, D), full),
            pl.BlockSpec((1, D), full),
            pl.BlockSpec((D, E), full),
        ],
        out_specs=[
            pl.BlockSpec((RB, D), row),
            pl.BlockSpec((RB, D), row),
            pl.BlockSpec((RB, E), row),
        ],
        out_shape=[
            jax.ShapeDtypeStruct((S, D), jnp.float32),
            jax.ShapeDtypeStruct((S, D), jnp.float32),
            jax.ShapeDtypeStruct((S, E), jnp.float32),
        ],
    )(x, ao, owT, ln2_w, gwT)


def _silu(a):
    return a * (1.0 / (1.0 + jnp.exp(-a)))


def _moe_body(nlive_ref, be_ref, xg_ref, eg_ref, eu_ref, ed_ref, yg_ref):
    @pl.when(pl.program_id(0) < nlive_ref[0])
    def _():
        xb = xg_ref[...].astype(jnp.bfloat16)
        a = lax.dot_general(xb, eg_ref[0], (((1,), (1,)), ((), ())),
                            preferred_element_type=jnp.float32)
        u = lax.dot_general(xb, eu_ref[0], (((1,), (1,)), ((), ())),
                            preferred_element_type=jnp.float32)
        s = (_silu(a) * u).astype(jnp.bfloat16)
        yg_ref[...] = lax.dot_general(s, ed_ref[0], (((1,), (1,)), ((), ())),
                                      preferred_element_type=jnp.float32)


def _grouped_ffn(xg, egb, eub, edb, be, nlive):
    grid_spec = pltpu.PrefetchScalarGridSpec(
        num_scalar_prefetch=2,
        grid=(NB,),
        in_specs=[
            pl.BlockSpec((BLK, D), lambda b, nl, be: (b, 0)),
            pl.BlockSpec((1, DFF, D), lambda b, nl, be: (be[b], 0, 0)),
            pl.BlockSpec((1, DFF, D), lambda b, nl, be: (be[b], 0, 0)),
            pl.BlockSpec((1, D, DFF), lambda b, nl, be: (be[b], 0, 0)),
        ],
        out_specs=pl.BlockSpec((BLK, D), lambda b, nl, be: (b, 0)),
    )
    return pl.pallas_call(
        _moe_body,
        grid_spec=grid_spec,
        out_shape=jax.ShapeDtypeStruct((NP_PAD, D), jnp.float32),
    )(nlive, be, xg, egb, eub, edb)


def _shared_body(h1_ref, x2_ref, sg_ref, su_ref, sd_ref, o_ref):
    xb = x2_ref[...].astype(jnp.bfloat16)
    a = jnp.dot(xb, sg_ref[...], preferred_element_type=jnp.float32)
    u = jnp.dot(xb, su_ref[...], preferred_element_type=jnp.float32)
    s = (_silu(a) * u).astype(jnp.bfloat16)
    o_ref[...] = h1_ref[...] + jnp.dot(s, sd_ref[...],
                                       preferred_element_type=jnp.float32)


def _shared_ffn(h1, x2, sgT, suT, sdT):
    row = lambda i: (i, 0)
    full = lambda i: (0, 0)
    return pl.pallas_call(
        _shared_body,
        grid=(S // RB,),
        in_specs=[
            pl.BlockSpec((RB, D), row),
            pl.BlockSpec((RB, D), row),
            pl.BlockSpec((D, SFF), full),
            pl.BlockSpec((D, SFF), full),
            pl.BlockSpec((SFF, D), full),
        ],
        out_specs=pl.BlockSpec((RB, D), row),
        out_shape=jax.ShapeDtypeStruct((S, D), jnp.float32),
    )(h1, x2, sgT, suT, sdT)


def _combine_body(base_ref, g0_ref, g1_ref, w0_ref, w1_ref, o_ref):
    o_ref[...] = (base_ref[...] + w0_ref[...] * g0_ref[...]
                  + w1_ref[...] * g1_ref[...])


def _combine(base, g0, g1, w0, w1):
    row = lambda i: (i, 0)
    return pl.pallas_call(
        _combine_body,
        grid=(S // RB,),
        in_specs=[
            pl.BlockSpec((RB, D), row),
            pl.BlockSpec((RB, D), row),
            pl.BlockSpec((RB, D), row),
            pl.BlockSpec((RB, 1), row),
            pl.BlockSpec((RB, 1), row),
        ],
        out_specs=pl.BlockSpec((RB, D), row),
        out_shape=jax.ShapeDtypeStruct((S, D), jnp.float32),
    )(base, g0, g1, w0, w1)


def _sc_gather_rows(table, idx, chunk=32, nbuf=3):
    info = plsc.get_sparse_core_info()
    nw = info.num_cores * info.num_subcores
    n, tail = idx.shape[0], table.shape[1:]
    per_w = n // nw
    n_ch = per_w // chunk
    assert n_ch * chunk == per_w
    mesh = plsc.VectorSubcoreMesh(core_axis_name="c", subcore_axis_name="s")

    @functools.partial(
        pl.kernel, mesh=mesh,
        out_type=jax.ShapeDtypeStruct((n,) + tail, jnp.float32),
        scratch_types=(
            [pltpu.VMEM((per_w,), jnp.int32)]
            + [pltpu.VMEM((chunk,) + tail, jnp.float32)] * nbuf
            + [pltpu.SemaphoreType.DMA] * (2 * nbuf)
        ),
    )
    def k(table_hbm, idx_hbm, out_hbm, idx_v, *bufs_sems):
        bufs = bufs_sems[:nbuf]
        gsems = bufs_sems[nbuf:2 * nbuf]
        wsems = bufs_sems[2 * nbuf:]
        wid = lax.axis_index("s") * info.num_cores + lax.axis_index("c")
        base = wid * per_w
        pltpu.sync_copy(idx_hbm.at[pl.ds(base, per_w)], idx_v)

        def start_gather(c, b):
            return pltpu.async_copy(
                table_hbm.at[idx_v.at[pl.ds(c * chunk, chunk)]],
                bufs[b], gsems[b])

        gh, wh = {}, {}
        for c in range(min(nbuf, n_ch)):
            gh[c] = start_gather(c, c % nbuf)
        for c in range(n_ch):
            b = c % nbuf
            gh[c].wait()
            wh[c] = pltpu.async_copy(
                bufs[b], out_hbm.at[pl.ds(base + c * chunk, chunk)], wsems[b])
            if c + nbuf < n_ch:
                wh[c].wait()
                gh[c + nbuf] = start_gather(c + nbuf, b)
        for c in range(max(0, n_ch - nbuf), n_ch):
            wh[c].wait()

    return k(table, idx)


def kernel(hidden_states, position_ids, ln1_w, q_w, k_w, v_w, o_w, ln2_w,
           gate_w, eg, eu, ed, sg, su, sd):
    x = hidden_states.reshape(S, D)

    inv_freq = 1.0 / (ROPE_BASE ** (jnp.arange(0, HD, 2, dtype=jnp.float32) / HD))
    freqs = jnp.outer(jnp.arange(S, dtype=jnp.float32), inv_freq)
    emb = jnp.concatenate([freqs, freqs], axis=-1)
    pos = position_ids.reshape(S)
    cosE = jnp.tile(jnp.cos(emb)[pos], (1, H))
    sinE = jnp.tile(jnp.sin(emb)[pos], (1, H))

    wqT = q_w.T.astype(jnp.bfloat16)
    wkT = k_w.T.astype(jnp.bfloat16)
    wvT = v_w.T.astype(jnp.bfloat16)
    q, k, v = _qkv_rope(x, cosE, sinE, ln1_w.reshape(1, D), wqT,
                        _rot_weight(wqT), wkT, _rot_weight(wkT), wvT)

    qh = q.reshape(S, H, HD).transpose(1, 0, 2)
    kh = k.reshape(S, H, HD).transpose(1, 0, 2)
    vh = v.reshape(S, H, HD).transpose(1, 0, 2)
    ao = _attention(qh, kh, vh).transpose(1, 0, 2).reshape(S, D)

    h1, x2, logits = _oproj_ln2_gate(x, ao, o_w.T.astype(jnp.bfloat16),
                                     ln2_w.reshape(1, D), gate_w.T)

    scores = jax.nn.softmax(logits, axis=-1)
    topk_w, topk_idx = jax.lax.top_k(scores, 2)
    e_flat = topk_idx.reshape(-1)
    onehot = (e_flat[:, None] == jnp.arange(E)[None, :]).astype(jnp.int32)
    csum = jnp.cumsum(onehot, axis=0) - onehot
    rank = jnp.take_along_axis(csum, e_flat[:, None], axis=1)[:, 0]
    cnt = onehot.sum(0)
    pc = ((cnt + BLK - 1) // BLK) * BLK
    ps = jnp.concatenate([jnp.zeros(1, jnp.int32),
                          jnp.cumsum(pc)[:-1].astype(jnp.int32)])
    dst = ps[e_flat] + rank
    gather_idx = (jnp.arange(NP_PAD, dtype=jnp.int32) % S).at[dst].set(
        jnp.arange(4096, dtype=jnp.int32) // 2)
    bpos = jnp.arange(NB, dtype=jnp.int32) * BLK
    ends = (ps + pc)[None, :]
    be = jnp.minimum(jnp.sum((bpos[:, None] >= ends).astype(jnp.int32),
                             axis=1), E - 1).astype(jnp.int32)
    nlive = jnp.array([0], jnp.int32) + (jnp.sum(pc) + BLK - 1) // BLK

    xg = _sc_gather_rows(x2, gather_idx)
    yg = _grouped_ffn(xg, eg.astype(jnp.bfloat16), eu.astype(jnp.bfloat16),
                      ed.astype(jnp.bfloat16), be, nlive)
    back_idx = jnp.concatenate([dst[0::2], dst[1::2]])
    gathered = _sc_gather_rows(yg, back_idx)
    g0 = gathered[:S]
    g1 = gathered[S:]

    base = _shared_ffn(h1, x2, sg.T.astype(jnp.bfloat16),
                       su.T.astype(jnp.bfloat16), sd.T.astype(jnp.bfloat16))
    out = _combine(base, g0, g1, topk_w[:, 0:1], topk_w[:, 1:2])
    return out.reshape(1, S, D)


# speedup vs baseline: 1.2191x; 1.0295x over previous
import functools

import jax
import jax.numpy as jnp
import numpy as np
from jax import lax
from jax.experimental import pallas as pl
from jax.experimental.pallas import tpu as pltpu
from jax.experimental.pallas import tpu_sc as plsc

S = 2048
D = 1024
H = 16
HD = 64
E = 8
DFF = 1408
SFF = 2816
EPS = 1e-6
ROPE_BASE = 10000.0

RB = 256
BLK = 256
NP_PAD = 4096 + 8 * (BLK - 1)
NP_PAD = ((NP_PAD + BLK - 1) // BLK) * BLK
NB = NP_PAD // BLK

def _rot_weight(wT):
    w4 = wT.reshape(D, H, 2, 32)
    return jnp.concatenate([-w4[:, :, 1], w4[:, :, 0]], axis=2).reshape(D, D)


def _k1_body(x_ref, cos_ref, sin_ref, ln1_ref, wq_ref, wqr_ref, wk_ref,
             wkr_ref, wv_ref, q_ref, k_ref, v_ref):
    x = x_ref[...]
    var = jnp.mean(x * x, axis=-1, keepdims=True)
    xn = ((x * lax.rsqrt(var + EPS)) * ln1_ref[...]).astype(jnp.bfloat16)
    c, s = cos_ref[...], sin_ref[...]
    q = jnp.dot(xn, wq_ref[...], preferred_element_type=jnp.float32)
    qr = jnp.dot(xn, wqr_ref[...], preferred_element_type=jnp.float32)
    q_ref[...] = (q * c + qr * s).astype(jnp.bfloat16)
    k = jnp.dot(xn, wk_ref[...], preferred_element_type=jnp.float32)
    kr = jnp.dot(xn, wkr_ref[...], preferred_element_type=jnp.float32)
    k_ref[...] = (k * c + kr * s).astype(jnp.bfloat16)
    v_ref[...] = jnp.dot(xn, wv_ref[...],
                         preferred_element_type=jnp.float32).astype(jnp.bfloat16)


def _qkv_rope(x, cosE, sinE, ln1_w, wqT, wqTR, wkT, wkTR, wvT):
    row = lambda i: (i, 0)
    full = lambda i: (0, 0)
    return pl.pallas_call(
        _k1_body,
        grid=(S // RB,),
        in_specs=[
            pl.BlockSpec((RB, D), row),
            pl.BlockSpec((RB, D), row),
            pl.BlockSpec((RB, D), row),
            pl.BlockSpec((1, D), full),
            pl.BlockSpec((D, D), full),
            pl.BlockSpec((D, D), full),
            pl.BlockSpec((D, D), full),
            pl.BlockSpec((D, D), full),
            pl.BlockSpec((D, D), full),
        ],
        out_specs=[pl.BlockSpec((RB, D), row)] * 3,
        out_shape=[jax.ShapeDtypeStruct((S, D), jnp.bfloat16)] * 3,
    )(x, cosE, sinE, ln1_w, wqT, wqTR, wkT, wkTR, wvT)


def _attn_body(kv_len, qb0, q_ref, k_ref, v_ref, o_ref):
    q = q_ref[0]
    k = k_ref[0]
    s = lax.dot_general(q, k, (((1,), (1,)), ((), ())),
                        preferred_element_type=jnp.float32) * (1.0 / 8.0)
    qb = pl.program_id(1) + qb0
    rows = qb * RB + lax.broadcasted_iota(jnp.int32, (RB, kv_len), 0)
    cols = lax.broadcasted_iota(jnp.int32, (RB, kv_len), 1)
    s = jnp.where(rows >= cols, s, -1e30)
    m = jnp.max(s, axis=-1, keepdims=True)
    p = jnp.exp(s - m)
    p = (p / jnp.sum(p, axis=-1, keepdims=True)).astype(jnp.bfloat16)
    o_ref[0] = lax.dot_general(p, v_ref[0], (((1,), (0,)), ((), ())),
                               preferred_element_type=jnp.float32
                               ).astype(jnp.bfloat16)


def _attention_part(qh, kh, vh, qb0, n_qb, kv_len):
    return pl.pallas_call(
        functools.partial(_attn_body, kv_len, qb0),
        grid=(H, n_qb),
        in_specs=[
            pl.BlockSpec((1, RB, HD), lambda h, qb: (h, qb + qb0, 0)),
            pl.BlockSpec((1, kv_len, HD), lambda h, qb: (h, 0, 0)),
            pl.BlockSpec((1, kv_len, HD), lambda h, qb: (h, 0, 0)),
        ],
        out_specs=pl.BlockSpec((1, RB, HD), lambda h, qb: (h, qb, 0)),
        out_shape=jax.ShapeDtypeStruct((H, n_qb * RB, HD), jnp.bfloat16),
    )(qh, kh, vh)


def _attention(qh, kh, vh):
    half = S // (2 * RB)
    lo = _attention_part(qh, kh, vh, 0, half, half * RB)
    hi = _attention_part(qh, kh, vh, half, half, S)
    return jnp.concatenate([lo, hi], axis=1)


def _k3_body(x_ref, ao_ref, ow_ref, ln2_ref, gw_ref, h1_ref, x2_ref,
             i0_ref, i1_ref, w0_ref, w1_ref):
    proj = jnp.dot(ao_ref[...], ow_ref[...], preferred_element_type=jnp.float32)
    h1 = x_ref[...] + proj
    h1_ref[...] = h1
    var = jnp.mean(h1 * h1, axis=-1, keepdims=True)
    x2 = (h1 * lax.rsqrt(var + EPS)) * ln2_ref[...]
    x2_ref[...] = x2
    lg = jnp.dot(x2, gw_ref[...], preferred_element_type=jnp.float32)
    eidx = lax.broadcasted_iota(jnp.int32, (RB, E), 1)
    m1 = jnp.max(lg, axis=-1, keepdims=True)
    i0 = jnp.min(jnp.where(lg == m1, eidx, E), axis=-1, keepdims=True)
    lg2 = jnp.where(eidx == i0, -jnp.inf, lg)
    m2 = jnp.max(lg2, axis=-1, keepdims=True)
    i1 = jnp.min(jnp.where(lg2 == m2, eidx, E), axis=-1, keepdims=True)
    z = jnp.sum(jnp.exp(lg - m1), axis=-1, keepdims=True)
    i0_ref[...] = i0
    i1_ref[...] = i1
    w0_ref[...] = 1.0 / z
    w1_ref[...] = jnp.exp(m2 - m1) / z


def _oproj_ln2_gate(x, ao, owT, ln2_w, gwT):
    row = lambda i: (i, 0)
    full = lambda i: (0, 0)
    return pl.pallas_call(
        _k3_body,
        grid=(S // RB,),
        in_specs=[
            pl.BlockSpec((RB, D), row),
            pl.BlockSpec((RB, D), row),
            pl.BlockSpec((D, D), full),
            pl.BlockSpec((1, D), full),
            pl.BlockSpec((D, E), full),
        ],
        out_specs=[
            pl.BlockSpec((RB, D), row),
            pl.BlockSpec((RB, D), row),
            pl.BlockSpec((RB, 1), row),
            pl.BlockSpec((RB, 1), row),
            pl.BlockSpec((RB, 1), row),
            pl.BlockSpec((RB, 1), row),
        ],
        out_shape=[
            jax.ShapeDtypeStruct((S, D), jnp.float32),
            jax.ShapeDtypeStruct((S, D), jnp.float32),
            jax.ShapeDtypeStruct((S, 1), jnp.int32),
            jax.ShapeDtypeStruct((S, 1), jnp.int32),
            jax.ShapeDtypeStruct((S, 1), jnp.float32),
            jax.ShapeDtypeStruct((S, 1), jnp.float32),
        ],
    )(x, ao, owT, ln2_w, gwT)


def _silu(a):
    return a * (1.0 / (1.0 + jnp.exp(-a)))


def _moe_body(nlive_ref, be_ref, xg_ref, eg_ref, eu_ref, ed_ref, yg_ref):
    @pl.when(pl.program_id(0) < nlive_ref[0])
    def _():
        xb = xg_ref[...].astype(jnp.bfloat16)
        a = lax.dot_general(xb, eg_ref[0], (((1,), (1,)), ((), ())),
                            preferred_element_type=jnp.float32)
        u = lax.dot_general(xb, eu_ref[0], (((1,), (1,)), ((), ())),
                            preferred_element_type=jnp.float32)
        s = (_silu(a) * u).astype(jnp.bfloat16)
        yg_ref[...] = lax.dot_general(s, ed_ref[0], (((1,), (1,)), ((), ())),
                                      preferred_element_type=jnp.float32)


def _grouped_ffn(xg, egb, eub, edb, be, nlive):
    grid_spec = pltpu.PrefetchScalarGridSpec(
        num_scalar_prefetch=2,
        grid=(NB,),
        in_specs=[
            pl.BlockSpec((BLK, D), lambda b, nl, be: (b, 0)),
            pl.BlockSpec((1, DFF, D), lambda b, nl, be: (be[b], 0, 0)),
            pl.BlockSpec((1, DFF, D), lambda b, nl, be: (be[b], 0, 0)),
            pl.BlockSpec((1, D, DFF), lambda b, nl, be: (be[b], 0, 0)),
        ],
        out_specs=pl.BlockSpec((BLK, D), lambda b, nl, be: (b, 0)),
    )
    return pl.pallas_call(
        _moe_body,
        grid_spec=grid_spec,
        out_shape=jax.ShapeDtypeStruct((NP_PAD, D), jnp.float32),
    )(nlive, be, xg, egb, eub, edb)


def _shared_body(h1_ref, x2_ref, sg_ref, su_ref, sd_ref, o_ref):
    xb = x2_ref[...].astype(jnp.bfloat16)
    a = jnp.dot(xb, sg_ref[...], preferred_element_type=jnp.float32)
    u = jnp.dot(xb, su_ref[...], preferred_element_type=jnp.float32)
    s = (_silu(a) * u).astype(jnp.bfloat16)
    o_ref[...] = h1_ref[...] + jnp.dot(s, sd_ref[...],
                                       preferred_element_type=jnp.float32)


def _shared_ffn(h1, x2, sgT, suT, sdT):
    row = lambda i: (i, 0)
    full = lambda i: (0, 0)
    return pl.pallas_call(
        _shared_body,
        grid=(S // RB,),
        in_specs=[
            pl.BlockSpec((RB, D), row),
            pl.BlockSpec((RB, D), row),
            pl.BlockSpec((D, SFF), full),
            pl.BlockSpec((D, SFF), full),
            pl.BlockSpec((SFF, D), full),
        ],
        out_specs=pl.BlockSpec((RB, D), row),
        out_shape=jax.ShapeDtypeStruct((S, D), jnp.float32),
    )(h1, x2, sgT, suT, sdT)


def _combine_body(base_ref, g0_ref, g1_ref, w0_ref, w1_ref, o_ref):
    o_ref[...] = (base_ref[...] + w0_ref[...] * g0_ref[...]
                  + w1_ref[...] * g1_ref[...])


def _combine(base, g0, g1, w0, w1):
    row = lambda i: (i, 0)
    return pl.pallas_call(
        _combine_body,
        grid=(S // RB,),
        in_specs=[
            pl.BlockSpec((RB, D), row),
            pl.BlockSpec((RB, D), row),
            pl.BlockSpec((RB, D), row),
            pl.BlockSpec((RB, 1), row),
            pl.BlockSpec((RB, 1), row),
        ],
        out_specs=pl.BlockSpec((RB, D), row),
        out_shape=jax.ShapeDtypeStruct((S, D), jnp.float32),
    )(base, g0, g1, w0, w1)


def _sc_gather_rows(table, idx, chunk=32, nbuf=3):
    info = plsc.get_sparse_core_info()
    nw = info.num_cores * info.num_subcores
    n, tail = idx.shape[0], table.shape[1:]
    per_w = n // nw
    n_ch = per_w // chunk
    assert n_ch * chunk == per_w
    mesh = plsc.VectorSubcoreMesh(core_axis_name="c", subcore_axis_name="s")

    @functools.partial(
        pl.kernel, mesh=mesh,
        out_type=jax.ShapeDtypeStruct((n,) + tail, jnp.float32),
        scratch_types=(
            [pltpu.VMEM((per_w,), jnp.int32)]
            + [pltpu.VMEM((chunk,) + tail, jnp.float32)] * nbuf
            + [pltpu.SemaphoreType.DMA] * (2 * nbuf)
        ),
    )
    def k(table_hbm, idx_hbm, out_hbm, idx_v, *bufs_sems):
        bufs = bufs_sems[:nbuf]
        gsems = bufs_sems[nbuf:2 * nbuf]
        wsems = bufs_sems[2 * nbuf:]
        wid = lax.axis_index("s") * info.num_cores + lax.axis_index("c")
        base = wid * per_w
        pltpu.sync_copy(idx_hbm.at[pl.ds(base, per_w)], idx_v)

        def start_gather(c, b):
            return pltpu.async_copy(
                table_hbm.at[idx_v.at[pl.ds(c * chunk, chunk)]],
                bufs[b], gsems[b])

        gh, wh = {}, {}
        for c in range(min(nbuf, n_ch)):
            gh[c] = start_gather(c, c % nbuf)
        for c in range(n_ch):
            b = c % nbuf
            gh[c].wait()
            wh[c] = pltpu.async_copy(
                bufs[b], out_hbm.at[pl.ds(base + c * chunk, chunk)], wsems[b])
            if c + nbuf < n_ch:
                wh[c].wait()
                gh[c + nbuf] = start_gather(c + nbuf, b)
        for c in range(max(0, n_ch - nbuf), n_ch):
            wh[c].wait()

    return k(table, idx)


def kernel(hidden_states, position_ids, ln1_w, q_w, k_w, v_w, o_w, ln2_w,
           gate_w, eg, eu, ed, sg, su, sd):
    x = hidden_states.reshape(S, D)

    inv_freq = 1.0 / (ROPE_BASE ** (jnp.arange(0, HD, 2, dtype=jnp.float32) / HD))
    freqs = jnp.outer(jnp.arange(S, dtype=jnp.float32), inv_freq)
    emb = jnp.concatenate([freqs, freqs], axis=-1)
    pos = position_ids.reshape(S)
    cosE = jnp.tile(jnp.cos(emb)[pos], (1, H))
    sinE = jnp.tile(jnp.sin(emb)[pos], (1, H))

    wqT = q_w.T.astype(jnp.bfloat16)
    wkT = k_w.T.astype(jnp.bfloat16)
    wvT = v_w.T.astype(jnp.bfloat16)
    q, k, v = _qkv_rope(x, cosE, sinE, ln1_w.reshape(1, D), wqT,
                        _rot_weight(wqT), wkT, _rot_weight(wkT), wvT)

    qh = q.reshape(S, H, HD).transpose(1, 0, 2)
    kh = k.reshape(S, H, HD).transpose(1, 0, 2)
    vh = v.reshape(S, H, HD).transpose(1, 0, 2)
    ao = _attention(qh, kh, vh).transpose(1, 0, 2).reshape(S, D)

    h1, x2, i0, i1, w0, w1 = _oproj_ln2_gate(x, ao,
                                             o_w.T.astype(jnp.bfloat16),
                                             ln2_w.reshape(1, D), gate_w.T)

    e_flat = jnp.concatenate([i0, i1], axis=1).reshape(-1)
    onehot = (e_flat[:, None] == jnp.arange(E)[None, :]).astype(jnp.int32)
    csum = jnp.cumsum(onehot, axis=0) - onehot
    rank = jnp.take_along_axis(csum, e_flat[:, None], axis=1)[:, 0]
    cnt = onehot.sum(0)
    pc = ((cnt + BLK - 1) // BLK) * BLK
    ps = jnp.concatenate([jnp.zeros(1, jnp.int32),
                          jnp.cumsum(pc)[:-1].astype(jnp.int32)])
    dst = ps[e_flat] + rank
    gather_idx = (jnp.arange(NP_PAD, dtype=jnp.int32) % S).at[dst].set(
        jnp.arange(4096, dtype=jnp.int32) // 2)
    bpos = jnp.arange(NB, dtype=jnp.int32) * BLK
    ends = (ps + pc)[None, :]
    be = jnp.minimum(jnp.sum((bpos[:, None] >= ends).astype(jnp.int32),
                             axis=1), E - 1).astype(jnp.int32)
    nlive = jnp.array([0], jnp.int32) + (jnp.sum(pc) + BLK - 1) // BLK

    xg = _sc_gather_rows(x2, gather_idx)
    yg = _grouped_ffn(xg, eg.astype(jnp.bfloat16), eu.astype(jnp.bfloat16),
                      ed.astype(jnp.bfloat16), be, nlive)
    back_idx = jnp.concatenate([dst[0::2], dst[1::2]])
    gathered = _sc_gather_rows(yg, back_idx)
    g0 = gathered[:S]
    g1 = gathered[S:]

    base = _shared_ffn(h1, x2, sg.T.astype(jnp.bfloat16),
                       su.T.astype(jnp.bfloat16), sd.T.astype(jnp.bfloat16))
    out = _combine(base, g0, g1, w0, w1)
    return out.reshape(1, S, D)


# speedup vs baseline: 1.2762x; 1.0468x over previous
import functools

import jax
import jax.numpy as jnp
import numpy as np
from jax import lax
from jax.experimental import pallas as pl
from jax.experimental.pallas import tpu as pltpu
from jax.experimental.pallas import tpu_sc as plsc

S = 2048
D = 1024
H = 16
HD = 64
E = 8
DFF = 1408
SFF = 2816
EPS = 1e-6
ROPE_BASE = 10000.0

RB = 256
BLK = 256
NP_PAD = 4096 + 8 * (BLK - 1)
NP_PAD = ((NP_PAD + BLK - 1) // BLK) * BLK
NB = NP_PAD // BLK

def _rot_weight(wT):
    w4 = wT.reshape(D, H, 2, 32)
    return jnp.concatenate([-w4[:, :, 1], w4[:, :, 0]], axis=2).reshape(D, D)


def _k1_body(x_ref, cos_ref, sin_ref, ln1_ref, wq_ref, wqr_ref, wk_ref,
             wkr_ref, wv_ref, q_ref, k_ref, v_ref):
    x = x_ref[...]
    var = jnp.mean(x * x, axis=-1, keepdims=True)
    xn = ((x * lax.rsqrt(var + EPS)) * ln1_ref[...]).astype(jnp.bfloat16)
    c, s = cos_ref[...], sin_ref[...]
    q = jnp.dot(xn, wq_ref[...], preferred_element_type=jnp.float32)
    qr = jnp.dot(xn, wqr_ref[...], preferred_element_type=jnp.float32)
    q_ref[...] = (q * c + qr * s).astype(jnp.bfloat16)
    k = jnp.dot(xn, wk_ref[...], preferred_element_type=jnp.float32)
    kr = jnp.dot(xn, wkr_ref[...], preferred_element_type=jnp.float32)
    k_ref[...] = (k * c + kr * s).astype(jnp.bfloat16)
    v_ref[...] = jnp.dot(xn, wv_ref[...],
                         preferred_element_type=jnp.float32).astype(jnp.bfloat16)


def _qkv_rope(x, cosE, sinE, ln1_w, wqT, wqTR, wkT, wkTR, wvT):
    row = lambda i: (i, 0)
    full = lambda i: (0, 0)
    return pl.pallas_call(
        _k1_body,
        grid=(S // RB,),
        in_specs=[
            pl.BlockSpec((RB, D), row),
            pl.BlockSpec((RB, D), row),
            pl.BlockSpec((RB, D), row),
            pl.BlockSpec((1, D), full),
            pl.BlockSpec((D, D), full),
            pl.BlockSpec((D, D), full),
            pl.BlockSpec((D, D), full),
            pl.BlockSpec((D, D), full),
            pl.BlockSpec((D, D), full),
        ],
        out_specs=[pl.BlockSpec((RB, D), row)] * 3,
        out_shape=[jax.ShapeDtypeStruct((S, D), jnp.bfloat16)] * 3,
    )(x, cosE, sinE, ln1_w, wqT, wqTR, wkT, wkTR, wvT)


def _attn_body(kv_len, qb0, q_ref, k_ref, v_ref, o_ref):
    q = q_ref[0]
    k = k_ref[0]
    s = lax.dot_general(q, k, (((1,), (1,)), ((), ())),
                        preferred_element_type=jnp.float32) * (1.0 / 8.0)
    qb = pl.program_id(1) + qb0
    rows = qb * RB + lax.broadcasted_iota(jnp.int32, (RB, kv_len), 0)
    cols = lax.broadcasted_iota(jnp.int32, (RB, kv_len), 1)
    s = jnp.where(rows >= cols, s, -1e30)
    m = jnp.max(s, axis=-1, keepdims=True)
    p = jnp.exp(s - m)
    p = (p / jnp.sum(p, axis=-1, keepdims=True)).astype(jnp.bfloat16)
    o_ref[0] = lax.dot_general(p, v_ref[0], (((1,), (0,)), ((), ())),
                               preferred_element_type=jnp.float32
                               ).astype(jnp.bfloat16)


def _attention_part(qh, kh, vh, qb0, n_qb, kv_len):
    return pl.pallas_call(
        functools.partial(_attn_body, kv_len, qb0),
        grid=(H, n_qb),
        in_specs=[
            pl.BlockSpec((1, RB, HD), lambda h, qb: (h, qb + qb0, 0)),
            pl.BlockSpec((1, kv_len, HD), lambda h, qb: (h, 0, 0)),
            pl.BlockSpec((1, kv_len, HD), lambda h, qb: (h, 0, 0)),
        ],
        out_specs=pl.BlockSpec((1, RB, HD), lambda h, qb: (h, qb, 0)),
        out_shape=jax.ShapeDtypeStruct((H, n_qb * RB, HD), jnp.bfloat16),
    )(qh, kh, vh)


def _attention(qh, kh, vh):
    half = S // (2 * RB)
    lo = _attention_part(qh, kh, vh, 0, half, half * RB)
    hi = _attention_part(qh, kh, vh, half, half, S)
    return jnp.concatenate([lo, hi], axis=1)


def _k3_body(x_ref, ao_ref, ow_ref, ln2_ref, gw_ref, h1_ref, x2_ref,
             i0_ref, i1_ref, w0_ref, w1_ref):
    proj = jnp.dot(ao_ref[...], ow_ref[...], preferred_element_type=jnp.float32)
    h1 = x_ref[...] + proj
    h1_ref[...] = h1
    var = jnp.mean(h1 * h1, axis=-1, keepdims=True)
    x2 = (h1 * lax.rsqrt(var + EPS)) * ln2_ref[...]
    x2_ref[...] = x2
    lg = jnp.dot(x2, gw_ref[...], preferred_element_type=jnp.float32)
    eidx = lax.broadcasted_iota(jnp.int32, (RB, E), 1)
    m1 = jnp.max(lg, axis=-1, keepdims=True)
    i0 = jnp.min(jnp.where(lg == m1, eidx, E), axis=-1, keepdims=True)
    lg2 = jnp.where(eidx == i0, -jnp.inf, lg)
    m2 = jnp.max(lg2, axis=-1, keepdims=True)
    i1 = jnp.min(jnp.where(lg2 == m2, eidx, E), axis=-1, keepdims=True)
    z = jnp.sum(jnp.exp(lg - m1), axis=-1, keepdims=True)
    i0_ref[...] = i0
    i1_ref[...] = i1
    w0_ref[...] = 1.0 / z
    w1_ref[...] = jnp.exp(m2 - m1) / z


def _oproj_ln2_gate(x, ao, owT, ln2_w, gwT):
    row = lambda i: (i, 0)
    full = lambda i: (0, 0)
    return pl.pallas_call(
        _k3_body,
        grid=(S // RB,),
        in_specs=[
            pl.BlockSpec((RB, D), row),
            pl.BlockSpec((RB, D), row),
            pl.BlockSpec((D, D), full),
            pl.BlockSpec((1, D), full),
            pl.BlockSpec((D, E), full),
        ],
        out_specs=[
            pl.BlockSpec((RB, D), row),
            pl.BlockSpec((RB, D), row),
            pl.BlockSpec((RB, 1), row),
            pl.BlockSpec((RB, 1), row),
            pl.BlockSpec((RB, 1), row),
            pl.BlockSpec((RB, 1), row),
        ],
        out_shape=[
            jax.ShapeDtypeStruct((S, D), jnp.float32),
            jax.ShapeDtypeStruct((S, D), jnp.float32),
            jax.ShapeDtypeStruct((S, 1), jnp.int32),
            jax.ShapeDtypeStruct((S, 1), jnp.int32),
            jax.ShapeDtypeStruct((S, 1), jnp.float32),
            jax.ShapeDtypeStruct((S, 1), jnp.float32),
        ],
    )(x, ao, owT, ln2_w, gwT)


def _silu(a):
    return a * (1.0 / (1.0 + jnp.exp(-a)))


def _moe_body(nlive_ref, be_ref, xg_ref, eg_ref, eu_ref, ed_ref, yg_ref):
    @pl.when(pl.program_id(0) < nlive_ref[0])
    def _():
        xb = xg_ref[...].astype(jnp.bfloat16)
        a = lax.dot_general(xb, eg_ref[0], (((1,), (1,)), ((), ())),
                            preferred_element_type=jnp.float32)
        u = lax.dot_general(xb, eu_ref[0], (((1,), (1,)), ((), ())),
                            preferred_element_type=jnp.float32)
        s = (_silu(a) * u).astype(jnp.bfloat16)
        yg_ref[...] = lax.dot_general(s, ed_ref[0], (((1,), (1,)), ((), ())),
                                      preferred_element_type=jnp.float32)


def _grouped_ffn(xg, egb, eub, edb, be, nlive):
    grid_spec = pltpu.PrefetchScalarGridSpec(
        num_scalar_prefetch=2,
        grid=(NB,),
        in_specs=[
            pl.BlockSpec((BLK, D), lambda b, nl, be: (b, 0)),
            pl.BlockSpec((1, DFF, D), lambda b, nl, be: (be[b], 0, 0)),
            pl.BlockSpec((1, DFF, D), lambda b, nl, be: (be[b], 0, 0)),
            pl.BlockSpec((1, D, DFF), lambda b, nl, be: (be[b], 0, 0)),
        ],
        out_specs=pl.BlockSpec((BLK, D), lambda b, nl, be: (b, 0)),
    )
    return pl.pallas_call(
        _moe_body,
        grid_spec=grid_spec,
        out_shape=jax.ShapeDtypeStruct((NP_PAD, D), jnp.float32),
    )(nlive, be, xg, egb, eub, edb)


def _shared_body(h1_ref, x2_ref, sg_ref, su_ref, sd_ref, o_ref):
    xb = x2_ref[...].astype(jnp.bfloat16)
    a = jnp.dot(xb, sg_ref[...], preferred_element_type=jnp.float32)
    u = jnp.dot(xb, su_ref[...], preferred_element_type=jnp.float32)
    s = (_silu(a) * u).astype(jnp.bfloat16)
    o_ref[...] = h1_ref[...] + jnp.dot(s, sd_ref[...],
                                       preferred_element_type=jnp.float32)


def _shared_ffn(h1, x2, sgT, suT, sdT):
    row = lambda i: (i, 0)
    full = lambda i: (0, 0)
    return pl.pallas_call(
        _shared_body,
        grid=(S // RB,),
        in_specs=[
            pl.BlockSpec((RB, D), row),
            pl.BlockSpec((RB, D), row),
            pl.BlockSpec((D, SFF), full),
            pl.BlockSpec((D, SFF), full),
            pl.BlockSpec((SFF, D), full),
        ],
        out_specs=pl.BlockSpec((RB, D), row),
        out_shape=jax.ShapeDtypeStruct((S, D), jnp.float32),
    )(h1, x2, sgT, suT, sdT)


def _combine_body(base_ref, g0_ref, g1_ref, w0_ref, w1_ref, o_ref):
    o_ref[...] = (base_ref[...] + w0_ref[...] * g0_ref[...]
                  + w1_ref[...] * g1_ref[...])


def _combine(base, g0, g1, w0, w1):
    row = lambda i: (i, 0)
    return pl.pallas_call(
        _combine_body,
        grid=(S // RB,),
        in_specs=[
            pl.BlockSpec((RB, D), row),
            pl.BlockSpec((RB, D), row),
            pl.BlockSpec((RB, D), row),
            pl.BlockSpec((RB, 1), row),
            pl.BlockSpec((RB, 1), row),
        ],
        out_specs=pl.BlockSpec((RB, D), row),
        out_shape=jax.ShapeDtypeStruct((S, D), jnp.float32),
    )(base, g0, g1, w0, w1)


def _sc_route(i0, i1):
    mesh = plsc.VectorSubcoreMesh(core_axis_name="c", subcore_axis_name="s")
    L = 16
    n_tok_ch = S // L

    @functools.partial(
        pl.kernel, mesh=mesh,
        out_type=(
            jax.ShapeDtypeStruct((2 * S,), jnp.int32),
            jax.ShapeDtypeStruct((32,), jnp.int32),
            jax.ShapeDtypeStruct((16,), jnp.int32),
        ),
        scratch_types=[
            pltpu.VMEM((S,), jnp.int32),
            pltpu.VMEM((S,), jnp.int32),
            pltpu.VMEM((2 * S,), jnp.int32),
            pltpu.VMEM((32,), jnp.int32),
            pltpu.VMEM((16,), jnp.int32),
        ],
    )
    def k(i0_hbm, i1_hbm, back_hbm, be_hbm, nl_hbm,
          i0_v, i1_v, back_v, be_v, nl_v):
        wid = lax.axis_index("s") * 2 + lax.axis_index("c")

        @pl.when(wid == 0)
        def _():
            pltpu.sync_copy(i0_hbm, i0_v)
            pltpu.sync_copy(i1_hbm, i1_v)
            lane = lax.iota(jnp.int32, L)
            zero = jnp.zeros((L,), jnp.int32)
            last = zero + (L - 1)
            dn = lax.GatherDimensionNumbers(offset_dims=(),
                                            collapsed_slice_dims=(0,),
                                            start_index_map=(0,))

            def perm(v, idx):
                return lax.gather(v, idx[:, None], dimension_numbers=dn,
                                  slice_sizes=(1,),
                                  mode=lax.GatherScatterMode.PROMISE_IN_BOUNDS)

            def incl_scan(v):
                for sh in (1, 2, 4, 8):
                    v = v + jnp.where(lane >= sh,
                                      perm(v, jnp.maximum(lane - sh, 0)), 0)
                return v

            def splat_last(v):
                return perm(v, last)

            def count_body(c, accs):
                e0 = i0_v[pl.ds(c * L, L)]
                e1 = i1_v[pl.ds(c * L, L)]
                return tuple(
                    accs[e]
                    + jnp.where(e0 == e, 1, 0)
                    + jnp.where(e1 == e, 1, 0)
                    for e in range(E))

            accs = lax.fori_loop(0, n_tok_ch, count_body, (zero,) * E)
            cnts = [splat_last(incl_scan(a)) for a in accs]
            pcs = [((c + BLK - 1) >> 8) << 8 for c in cnts]
            pss = [zero]
            for e in range(E):
                pss.append(pss[e] + pcs[e])
            nl_v[...] = pss[E] >> 8
            for c in range(2):
                bpos = (c * L + lane) * BLK
                acc = zero
                for e in range(E):
                    acc = acc + jnp.where(bpos >= pss[e] + pcs[e], 1, 0)
                be_v[pl.ds(c * L, L)] = jnp.minimum(acc, E - 1)

            def place_body(c, offs):
                for kk, ref in ((0, i0_v), (1, i1_v)):
                    e_vec = ref[pl.ds(c * L, L)]
                    dst = zero
                    new_offs = []
                    for e in range(E):
                        mi = jnp.where(e_vec == e, 1, 0)
                        inc = incl_scan(mi)
                        dst = jnp.where(e_vec == e, offs[e] + inc - mi, dst)
                        new_offs.append(offs[e] + splat_last(inc))
                    offs = tuple(new_offs)
                    back_v[pl.ds(kk * S + c * L, L)] = dst
                return offs

            lax.fori_loop(0, n_tok_ch, place_body, tuple(pss[:E]))

            pltpu.sync_copy(back_v, back_hbm)
            pltpu.sync_copy(be_v, be_hbm)
            pltpu.sync_copy(nl_v, nl_hbm)

    return k(i0, i1)


def _sc_scatter_rows(table, idx, out_rows, chunk=32):
    info = plsc.get_sparse_core_info()
    nw = info.num_cores * info.num_subcores
    n_tok, d = table.shape
    per_w = n_tok // nw
    n_ch = per_w // chunk
    mesh = plsc.VectorSubcoreMesh(core_axis_name="c", subcore_axis_name="s")

    @functools.partial(
        pl.kernel, mesh=mesh,
        out_type=jax.ShapeDtypeStruct((out_rows, d), jnp.float32),
        scratch_types=[
            pltpu.VMEM((chunk,), jnp.int32),
            pltpu.VMEM((chunk, d), jnp.float32),
            pltpu.SemaphoreType.DMA,
        ],
    )
    def k(table_hbm, idx_hbm, out_hbm, idx_c, buf, sem):
        wid = lax.axis_index("s") * info.num_cores + lax.axis_index("c")
        tbase = wid * per_w
        for c in range(n_ch):
            pltpu.sync_copy(table_hbm.at[pl.ds(tbase + c * chunk, chunk)], buf)
            for kk in range(2):
                pltpu.sync_copy(
                    idx_hbm.at[pl.ds(kk * S + tbase + c * chunk, chunk)],
                    idx_c)
                pltpu.async_copy(buf, out_hbm.at[idx_c], sem).wait()

    return k(table, idx)


def _sc_gather_rows(table, idx, chunk=32, nbuf=3):
    info = plsc.get_sparse_core_info()
    nw = info.num_cores * info.num_subcores
    n, tail = idx.shape[0], table.shape[1:]
    per_w = n // nw
    n_ch = per_w // chunk
    assert n_ch * chunk == per_w
    mesh = plsc.VectorSubcoreMesh(core_axis_name="c", subcore_axis_name="s")

    @functools.partial(
        pl.kernel, mesh=mesh,
        out_type=jax.ShapeDtypeStruct((n,) + tail, jnp.float32),
        scratch_types=(
            [pltpu.VMEM((per_w,), jnp.int32)]
            + [pltpu.VMEM((chunk,) + tail, jnp.float32)] * nbuf
            + [pltpu.SemaphoreType.DMA] * (2 * nbuf)
        ),
    )
    def k(table_hbm, idx_hbm, out_hbm, idx_v, *bufs_sems):
        bufs = bufs_sems[:nbuf]
        gsems = bufs_sems[nbuf:2 * nbuf]
        wsems = bufs_sems[2 * nbuf:]
        wid = lax.axis_index("s") * info.num_cores + lax.axis_index("c")
        base = wid * per_w
        pltpu.sync_copy(idx_hbm.at[pl.ds(base, per_w)], idx_v)

        def start_gather(c, b):
            return pltpu.async_copy(
                table_hbm.at[idx_v.at[pl.ds(c * chunk, chunk)]],
                bufs[b], gsems[b])

        gh, wh = {}, {}
        for c in range(min(nbuf, n_ch)):
            gh[c] = start_gather(c, c % nbuf)
        for c in range(n_ch):
            b = c % nbuf
            gh[c].wait()
            wh[c] = pltpu.async_copy(
                bufs[b], out_hbm.at[pl.ds(base + c * chunk, chunk)], wsems[b])
            if c + nbuf < n_ch:
                wh[c].wait()
                gh[c + nbuf] = start_gather(c + nbuf, b)
        for c in range(max(0, n_ch - nbuf), n_ch):
            wh[c].wait()

    return k(table, idx)


def kernel(hidden_states, position_ids, ln1_w, q_w, k_w, v_w, o_w, ln2_w,
           gate_w, eg, eu, ed, sg, su, sd):
    x = hidden_states.reshape(S, D)

    inv_freq = 1.0 / (ROPE_BASE ** (jnp.arange(0, HD, 2, dtype=jnp.float32) / HD))
    freqs = jnp.outer(jnp.arange(S, dtype=jnp.float32), inv_freq)
    emb = jnp.concatenate([freqs, freqs], axis=-1)
    pos = position_ids.reshape(S)
    cosE = jnp.tile(jnp.cos(emb)[pos], (1, H))
    sinE = jnp.tile(jnp.sin(emb)[pos], (1, H))

    wqT = q_w.T.astype(jnp.bfloat16)
    wkT = k_w.T.astype(jnp.bfloat16)
    wvT = v_w.T.astype(jnp.bfloat16)
    q, k, v = _qkv_rope(x, cosE, sinE, ln1_w.reshape(1, D), wqT,
                        _rot_weight(wqT), wkT, _rot_weight(wkT), wvT)

    qh = q.reshape(S, H, HD).transpose(1, 0, 2)
    kh = k.reshape(S, H, HD).transpose(1, 0, 2)
    vh = v.reshape(S, H, HD).transpose(1, 0, 2)
    ao = _attention(qh, kh, vh).transpose(1, 0, 2).reshape(S, D)

    h1, x2, i0, i1, w0, w1 = _oproj_ln2_gate(x, ao,
                                             o_w.T.astype(jnp.bfloat16),
                                             ln2_w.reshape(1, D), gate_w.T)

    back_idx, be, nlive = _sc_route(i0.reshape(S), i1.reshape(S))
    xg = _sc_scatter_rows(x2, back_idx, NP_PAD)
    yg = _grouped_ffn(xg, eg.astype(jnp.bfloat16), eu.astype(jnp.bfloat16),
                      ed.astype(jnp.bfloat16), be, nlive)
    gathered = _sc_gather_rows(yg, back_idx)
    g0 = gathered[:S]
    g1 = gathered[S:]

    base = _shared_ffn(h1, x2, sg.T.astype(jnp.bfloat16),
                       su.T.astype(jnp.bfloat16), sd.T.astype(jnp.bfloat16))
    out = _combine(base, g0, g1, w0, w1)
    return out.reshape(1, S, D)


# speedup vs baseline: 1.3161x; 1.0312x over previous
import functools

import jax
import jax.numpy as jnp
import numpy as np
from jax import lax
from jax.experimental import pallas as pl
from jax.experimental.pallas import tpu as pltpu
from jax.experimental.pallas import tpu_sc as plsc

S = 2048
D = 1024
H = 16
HD = 64
E = 8
DFF = 1408
SFF = 2816
EPS = 1e-6
ROPE_BASE = 10000.0

RB = 256
BLK = 256
NP_PAD = 4096 + 8 * (BLK - 1)
NP_PAD = ((NP_PAD + BLK - 1) // BLK) * BLK
NB = NP_PAD // BLK

def _rot_weight(wT):
    w4 = wT.reshape(D, H, 2, 32)
    return jnp.concatenate([-w4[:, :, 1], w4[:, :, 0]], axis=2).reshape(D, D)


def _k1_body(x_ref, cos_ref, sin_ref, ln1_ref, wq_ref, wqr_ref, wk_ref,
             wkr_ref, wv_ref, q_ref, k_ref, v_ref):
    x = x_ref[...]
    var = jnp.mean(x * x, axis=-1, keepdims=True)
    xn = ((x * lax.rsqrt(var + EPS)) * ln1_ref[...]).astype(jnp.bfloat16)
    c, s = cos_ref[...], sin_ref[...]
    q = jnp.dot(xn, wq_ref[...], preferred_element_type=jnp.float32)
    qr = jnp.dot(xn, wqr_ref[...], preferred_element_type=jnp.float32)
    q_ref[...] = (q * c + qr * s).astype(jnp.bfloat16)
    k = jnp.dot(xn, wk_ref[...], preferred_element_type=jnp.float32)
    kr = jnp.dot(xn, wkr_ref[...], preferred_element_type=jnp.float32)
    k_ref[...] = (k * c + kr * s).astype(jnp.bfloat16)
    v_ref[...] = jnp.dot(xn, wv_ref[...],
                         preferred_element_type=jnp.float32).astype(jnp.bfloat16)


def _qkv_rope(x, cosE, sinE, ln1_w, wqT, wqTR, wkT, wkTR, wvT):
    row = lambda i: (i, 0)
    full = lambda i: (0, 0)
    return pl.pallas_call(
        _k1_body,
        grid=(S // RB,),
        in_specs=[
            pl.BlockSpec((RB, D), row),
            pl.BlockSpec((RB, D), row),
            pl.BlockSpec((RB, D), row),
            pl.BlockSpec((1, D), full),
            pl.BlockSpec((D, D), full),
            pl.BlockSpec((D, D), full),
            pl.BlockSpec((D, D), full),
            pl.BlockSpec((D, D), full),
            pl.BlockSpec((D, D), full),
        ],
        out_specs=[pl.BlockSpec((RB, D), row)] * 3,
        out_shape=[jax.ShapeDtypeStruct((S, D), jnp.bfloat16)] * 3,
    )(x, cosE, sinE, ln1_w, wqT, wqTR, wkT, wkTR, wvT)


def _attn_body(kv_len, qb0, q_ref, k_ref, v_ref, o_ref):
    q = q_ref[0]
    k = k_ref[0]
    s = lax.dot_general(q, k, (((1,), (1,)), ((), ())),
                        preferred_element_type=jnp.float32) * (1.0 / 8.0)
    qb = pl.program_id(1) + qb0
    rows = qb * RB + lax.broadcasted_iota(jnp.int32, (RB, kv_len), 0)
    cols = lax.broadcasted_iota(jnp.int32, (RB, kv_len), 1)
    s = jnp.where(rows >= cols, s, -1e30)
    m = jnp.max(s, axis=-1, keepdims=True)
    p = jnp.exp(s - m)
    p = (p / jnp.sum(p, axis=-1, keepdims=True)).astype(jnp.bfloat16)
    o_ref[0] = lax.dot_general(p, v_ref[0], (((1,), (0,)), ((), ())),
                               preferred_element_type=jnp.float32
                               ).astype(jnp.bfloat16)


def _attention_part(qh, kh, vh, qb0, n_qb, kv_len):
    return pl.pallas_call(
        functools.partial(_attn_body, kv_len, qb0),
        grid=(H, n_qb),
        in_specs=[
            pl.BlockSpec((1, RB, HD), lambda h, qb: (h, qb + qb0, 0)),
            pl.BlockSpec((1, kv_len, HD), lambda h, qb: (h, 0, 0)),
            pl.BlockSpec((1, kv_len, HD), lambda h, qb: (h, 0, 0)),
        ],
        out_specs=pl.BlockSpec((1, RB, HD), lambda h, qb: (h, qb, 0)),
        out_shape=jax.ShapeDtypeStruct((H, n_qb * RB, HD), jnp.bfloat16),
    )(qh, kh, vh)


def _attention(qh, kh, vh):
    nq = S // RB
    step = nq // 4
    parts = [
        _attention_part(qh, kh, vh, i * step, step, (i + 1) * step * RB)
        for i in range(4)
    ]
    return jnp.concatenate(parts, axis=1)


def _k3_body(x_ref, ao_ref, ow_ref, ln2_ref, gw_ref, h1_ref, x2_ref,
             i0_ref, i1_ref, w0_ref, w1_ref):
    proj = jnp.dot(ao_ref[...], ow_ref[...], preferred_element_type=jnp.float32)
    h1 = x_ref[...] + proj
    h1_ref[...] = h1
    var = jnp.mean(h1 * h1, axis=-1, keepdims=True)
    x2 = (h1 * lax.rsqrt(var + EPS)) * ln2_ref[...]
    x2_ref[...] = x2
    lg = jnp.dot(x2, gw_ref[...], preferred_element_type=jnp.float32)
    eidx = lax.broadcasted_iota(jnp.int32, (RB, E), 1)
    m1 = jnp.max(lg, axis=-1, keepdims=True)
    i0 = jnp.min(jnp.where(lg == m1, eidx, E), axis=-1, keepdims=True)
    lg2 = jnp.where(eidx == i0, -jnp.inf, lg)
    m2 = jnp.max(lg2, axis=-1, keepdims=True)
    i1 = jnp.min(jnp.where(lg2 == m2, eidx, E), axis=-1, keepdims=True)
    z = jnp.sum(jnp.exp(lg - m1), axis=-1, keepdims=True)
    i0_ref[...] = i0
    i1_ref[...] = i1
    w0_ref[...] = 1.0 / z
    w1_ref[...] = jnp.exp(m2 - m1) / z


def _oproj_ln2_gate(x, ao, owT, ln2_w, gwT):
    row = lambda i: (i, 0)
    full = lambda i: (0, 0)
    return pl.pallas_call(
        _k3_body,
        grid=(S // RB,),
        in_specs=[
            pl.BlockSpec((RB, D), row),
            pl.BlockSpec((RB, D), row),
            pl.BlockSpec((D, D), full),
            pl.BlockSpec((1, D), full),
            pl.BlockSpec((D, E), full),
        ],
        out_specs=[
            pl.BlockSpec((RB, D), row),
            pl.BlockSpec((RB, D), row),
            pl.BlockSpec((RB, 1), row),
            pl.BlockSpec((RB, 1), row),
            pl.BlockSpec((RB, 1), row),
            pl.BlockSpec((RB, 1), row),
        ],
        out_shape=[
            jax.ShapeDtypeStruct((S, D), jnp.float32),
            jax.ShapeDtypeStruct((S, D), jnp.float32),
            jax.ShapeDtypeStruct((S, 1), jnp.int32),
            jax.ShapeDtypeStruct((S, 1), jnp.int32),
            jax.ShapeDtypeStruct((S, 1), jnp.float32),
            jax.ShapeDtypeStruct((S, 1), jnp.float32),
        ],
    )(x, ao, owT, ln2_w, gwT)


def _silu(a):
    return a * (1.0 / (1.0 + jnp.exp(-a)))


def _moe_body(nlive_ref, be_ref, xg_ref, eg_ref, eu_ref, ed_ref, yg_ref):
    @pl.when(pl.program_id(0) < nlive_ref[0])
    def _():
        xb = xg_ref[...].astype(jnp.bfloat16)
        a = lax.dot_general(xb, eg_ref[0], (((1,), (1,)), ((), ())),
                            preferred_element_type=jnp.float32)
        u = lax.dot_general(xb, eu_ref[0], (((1,), (1,)), ((), ())),
                            preferred_element_type=jnp.float32)
        s = (_silu(a) * u).astype(jnp.bfloat16)
        yg_ref[...] = lax.dot_general(s, ed_ref[0], (((1,), (1,)), ((), ())),
                                      preferred_element_type=jnp.float32)


def _grouped_ffn(xg, egb, eub, edb, be, nlive):
    grid_spec = pltpu.PrefetchScalarGridSpec(
        num_scalar_prefetch=2,
        grid=(NB,),
        in_specs=[
            pl.BlockSpec((BLK, D), lambda b, nl, be: (b, 0)),
            pl.BlockSpec((1, DFF, D), lambda b, nl, be: (be[b], 0, 0)),
            pl.BlockSpec((1, DFF, D), lambda b, nl, be: (be[b], 0, 0)),
            pl.BlockSpec((1, D, DFF), lambda b, nl, be: (be[b], 0, 0)),
        ],
        out_specs=pl.BlockSpec((BLK, D), lambda b, nl, be: (b, 0)),
    )
    return pl.pallas_call(
        _moe_body,
        grid_spec=grid_spec,
        out_shape=jax.ShapeDtypeStruct((NP_PAD, D), jnp.float32),
    )(nlive, be, xg, egb, eub, edb)


def _shared_body(h1_ref, x2_ref, sg_ref, su_ref, sd_ref, o_ref):
    xb = x2_ref[...].astype(jnp.bfloat16)
    a = jnp.dot(xb, sg_ref[...], preferred_element_type=jnp.float32)
    u = jnp.dot(xb, su_ref[...], preferred_element_type=jnp.float32)
    s = (_silu(a) * u).astype(jnp.bfloat16)
    o_ref[...] = h1_ref[...] + jnp.dot(s, sd_ref[...],
                                       preferred_element_type=jnp.float32)


def _shared_ffn(h1, x2, sgT, suT, sdT):
    row = lambda i: (i, 0)
    full = lambda i: (0, 0)
    return pl.pallas_call(
        _shared_body,
        grid=(S // RB,),
        in_specs=[
            pl.BlockSpec((RB, D), row),
            pl.BlockSpec((RB, D), row),
            pl.BlockSpec((D, SFF), full),
            pl.BlockSpec((D, SFF), full),
            pl.BlockSpec((SFF, D), full),
        ],
        out_specs=pl.BlockSpec((RB, D), row),
        out_shape=jax.ShapeDtypeStruct((S, D), jnp.float32),
    )(h1, x2, sgT, suT, sdT)


def _combine_body(base_ref, g0_ref, g1_ref, w0_ref, w1_ref, o_ref):
    o_ref[...] = (base_ref[...] + w0_ref[...] * g0_ref[...]
                  + w1_ref[...] * g1_ref[...])


def _combine(base, gathered, w0, w1):
    row = lambda i: (i, 0)
    nq = S // RB
    return pl.pallas_call(
        _combine_body,
        grid=(nq,),
        in_specs=[
            pl.BlockSpec((RB, D), row),
            pl.BlockSpec((RB, D), row),
            pl.BlockSpec((RB, D), lambda i: (i + nq, 0)),
            pl.BlockSpec((RB, 1), row),
            pl.BlockSpec((RB, 1), row),
        ],
        out_specs=pl.BlockSpec((RB, D), row),
        out_shape=jax.ShapeDtypeStruct((S, D), jnp.float32),
    )(base, gathered, gathered, w0, w1)


def _sc_route(i0, i1):
    mesh = plsc.VectorSubcoreMesh(core_axis_name="c", subcore_axis_name="s")
    L = 16
    n_tok_ch = S // L

    @functools.partial(
        pl.kernel, mesh=mesh,
        out_type=(
            jax.ShapeDtypeStruct((2 * S,), jnp.int32),
            jax.ShapeDtypeStruct((32,), jnp.int32),
            jax.ShapeDtypeStruct((16,), jnp.int32),
        ),
        scratch_types=[
            pltpu.VMEM((S,), jnp.int32),
            pltpu.VMEM((S,), jnp.int32),
            pltpu.VMEM((2 * S,), jnp.int32),
            pltpu.VMEM((32,), jnp.int32),
            pltpu.VMEM((16,), jnp.int32),
        ],
    )
    def k(i0_hbm, i1_hbm, back_hbm, be_hbm, nl_hbm,
          i0_v, i1_v, back_v, be_v, nl_v):
        wid = lax.axis_index("s") * 2 + lax.axis_index("c")

        @pl.when(wid == 0)
        def _():
            pltpu.sync_copy(i0_hbm, i0_v)
            pltpu.sync_copy(i1_hbm, i1_v)
            lane = lax.iota(jnp.int32, L)
            zero = jnp.zeros((L,), jnp.int32)
            last = zero + (L - 1)
            dn = lax.GatherDimensionNumbers(offset_dims=(),
                                            collapsed_slice_dims=(0,),
                                            start_index_map=(0,))

            def perm(v, idx):
                return lax.gather(v, idx[:, None], dimension_numbers=dn,
                                  slice_sizes=(1,),
                                  mode=lax.GatherScatterMode.PROMISE_IN_BOUNDS)

            def incl_scan(v):
                for sh in (1, 2, 4, 8):
                    v = v + jnp.where(lane >= sh,
                                      perm(v, jnp.maximum(lane - sh, 0)), 0)
                return v

            def splat_last(v):
                return perm(v, last)

            def count_body(c, accs):
                e0 = i0_v[pl.ds(c * L, L)]
                e1 = i1_v[pl.ds(c * L, L)]
                return tuple(
                    accs[e]
                    + jnp.where(e0 == e, 1, 0)
                    + jnp.where(e1 == e, 1, 0)
                    for e in range(E))

            accs = lax.fori_loop(0, n_tok_ch, count_body, (zero,) * E)
            cnts = [splat_last(incl_scan(a)) for a in accs]
            pcs = [((c + BLK - 1) >> 8) << 8 for c in cnts]
            pss = [zero]
            for e in range(E):
                pss.append(pss[e] + pcs[e])
            nl_v[...] = pss[E] >> 8
            for c in range(2):
                bpos = (c * L + lane) * BLK
                acc = zero
                for e in range(E):
                    acc = acc + jnp.where(bpos >= pss[e] + pcs[e], 1, 0)
                be_v[pl.ds(c * L, L)] = jnp.minimum(acc, E - 1)

            def place_body(c, offs):
                for kk, ref in ((0, i0_v), (1, i1_v)):
                    e_vec = ref[pl.ds(c * L, L)]
                    dst = zero
                    new_offs = []
                    for e in range(E):
                        mi = jnp.where(e_vec == e, 1, 0)
                        inc = incl_scan(mi)
                        dst = jnp.where(e_vec == e, offs[e] + inc - mi, dst)
                        new_offs.append(offs[e] + splat_last(inc))
                    offs = tuple(new_offs)
                    back_v[pl.ds(kk * S + c * L, L)] = dst
                return offs

            lax.fori_loop(0, n_tok_ch, place_body, tuple(pss[:E]))

            pltpu.sync_copy(back_v, back_hbm)
            pltpu.sync_copy(be_v, be_hbm)
            pltpu.sync_copy(nl_v, nl_hbm)

    return k(i0, i1)


def _sc_scatter_rows(table, idx, out_rows):
    info = plsc.get_sparse_core_info()
    nw = info.num_cores * info.num_subcores
    n_tok, d = table.shape
    per_w = n_tok // nw
    mesh = plsc.VectorSubcoreMesh(core_axis_name="c", subcore_axis_name="s")

    @functools.partial(
        pl.kernel, mesh=mesh,
        out_type=jax.ShapeDtypeStruct((out_rows, d), jnp.float32),
        scratch_types=[
            pltpu.VMEM((per_w,), jnp.int32),
            pltpu.VMEM((per_w,), jnp.int32),
            pltpu.VMEM((per_w, d), jnp.float32),
            pltpu.SemaphoreType.DMA,
            pltpu.SemaphoreType.DMA,
        ],
    )
    def k(table_hbm, idx_hbm, out_hbm, idx0_v, idx1_v, buf, sem0, sem1):
        wid = lax.axis_index("s") * info.num_cores + lax.axis_index("c")
        tbase = wid * per_w
        pltpu.sync_copy(idx_hbm.at[pl.ds(tbase, per_w)], idx0_v)
        pltpu.sync_copy(idx_hbm.at[pl.ds(S + tbase, per_w)], idx1_v)
        pltpu.sync_copy(table_hbm.at[pl.ds(tbase, per_w)], buf)
        h0 = pltpu.async_copy(buf, out_hbm.at[idx0_v], sem0)
        h1 = pltpu.async_copy(buf, out_hbm.at[idx1_v], sem1)
        h0.wait()
        h1.wait()

    return k(table, idx)


def _sc_gather_rows(table, idx, chunk=32, nbuf=3):
    info = plsc.get_sparse_core_info()
    nw = info.num_cores * info.num_subcores
    n, tail = idx.shape[0], table.shape[1:]
    per_w = n // nw
    n_ch = per_w // chunk
    assert n_ch * chunk == per_w
    mesh = plsc.VectorSubcoreMesh(core_axis_name="c", subcore_axis_name="s")

    @functools.partial(
        pl.kernel, mesh=mesh,
        out_type=jax.ShapeDtypeStruct((n,) + tail, jnp.float32),
        scratch_types=(
            [pltpu.VMEM((per_w,), jnp.int32)]
            + [pltpu.VMEM((chunk,) + tail, jnp.float32)] * nbuf
            + [pltpu.SemaphoreType.DMA] * (2 * nbuf)
        ),
    )
    def k(table_hbm, idx_hbm, out_hbm, idx_v, *bufs_sems):
        bufs = bufs_sems[:nbuf]
        gsems = bufs_sems[nbuf:2 * nbuf]
        wsems = bufs_sems[2 * nbuf:]
        wid = lax.axis_index("s") * info.num_cores + lax.axis_index("c")
        base = wid * per_w
        pltpu.sync_copy(idx_hbm.at[pl.ds(base, per_w)], idx_v)

        def start_gather(c, b):
            return pltpu.async_copy(
                table_hbm.at[idx_v.at[pl.ds(c * chunk, chunk)]],
                bufs[b], gsems[b])

        gh, wh = {}, {}
        for c in range(min(nbuf, n_ch)):
            gh[c] = start_gather(c, c % nbuf)
        for c in range(n_ch):
            b = c % nbuf
            gh[c].wait()
            wh[c] = pltpu.async_copy(
                bufs[b], out_hbm.at[pl.ds(base + c * chunk, chunk)], wsems[b])
            if c + nbuf < n_ch:
                wh[c].wait()
                gh[c + nbuf] = start_gather(c + nbuf, b)
        for c in range(max(0, n_ch - nbuf), n_ch):
            wh[c].wait()

    return k(table, idx)


def kernel(hidden_states, position_ids, ln1_w, q_w, k_w, v_w, o_w, ln2_w,
           gate_w, eg, eu, ed, sg, su, sd):
    x = hidden_states.reshape(S, D)

    inv_freq = 1.0 / (ROPE_BASE ** (jnp.arange(0, HD, 2, dtype=jnp.float32) / HD))
    freqs = jnp.outer(jnp.arange(S, dtype=jnp.float32), inv_freq)
    emb = jnp.concatenate([freqs, freqs], axis=-1)
    pos = position_ids.reshape(S)
    cosE = jnp.tile(jnp.cos(emb)[pos], (1, H))
    sinE = jnp.tile(jnp.sin(emb)[pos], (1, H))

    wqT = q_w.T.astype(jnp.bfloat16)
    wkT = k_w.T.astype(jnp.bfloat16)
    wvT = v_w.T.astype(jnp.bfloat16)
    q, k, v = _qkv_rope(x, cosE, sinE, ln1_w.reshape(1, D), wqT,
                        _rot_weight(wqT), wkT, _rot_weight(wkT), wvT)

    qh = q.reshape(S, H, HD).transpose(1, 0, 2)
    kh = k.reshape(S, H, HD).transpose(1, 0, 2)
    vh = v.reshape(S, H, HD).transpose(1, 0, 2)
    ao = _attention(qh, kh, vh).transpose(1, 0, 2).reshape(S, D)

    h1, x2, i0, i1, w0, w1 = _oproj_ln2_gate(x, ao,
                                             o_w.T.astype(jnp.bfloat16),
                                             ln2_w.reshape(1, D), gate_w.T)

    back_idx, be, nlive = _sc_route(i0.reshape(S), i1.reshape(S))
    xg = _sc_scatter_rows(x2, back_idx, NP_PAD)
    yg = _grouped_ffn(xg, eg.astype(jnp.bfloat16), eu.astype(jnp.bfloat16),
                      ed.astype(jnp.bfloat16), be, nlive)
    gathered = _sc_gather_rows(yg, back_idx)

    base = _shared_ffn(h1, x2, sg.T.astype(jnp.bfloat16),
                       su.T.astype(jnp.bfloat16), sd.T.astype(jnp.bfloat16))
    out = _combine(base, gathered, w0, w1)
    return out.reshape(1, S, D)


# speedup vs baseline: 1.3657x; 1.0377x over previous
import functools

import jax
import jax.numpy as jnp
import numpy as np
from jax import lax
from jax.experimental import pallas as pl
from jax.experimental.pallas import tpu as pltpu
from jax.experimental.pallas import tpu_sc as plsc

S = 2048
D = 1024
H = 16
HD = 64
E = 8
DFF = 1408
SFF = 2816
EPS = 1e-6
ROPE_BASE = 10000.0

RB = 256
BLK = 256
NP_PAD = 4096 + 8 * (BLK - 1)
NP_PAD = ((NP_PAD + BLK - 1) // BLK) * BLK
NB = NP_PAD // BLK

def _rotate_half_flat(x):
    first = (lax.broadcasted_iota(jnp.int32, x.shape, 1) & 63) < 32
    return jnp.where(first, -jnp.roll(x, -32, axis=1), jnp.roll(x, 32, axis=1))


def _k1_body(x_ref, cos_ref, sin_ref, ln1_ref, wq_ref, wk_ref,
             wv_ref, q_ref, k_ref, v_ref):
    x = x_ref[...]
    var = jnp.mean(x * x, axis=-1, keepdims=True)
    xn = ((x * lax.rsqrt(var + EPS)) * ln1_ref[...]).astype(jnp.bfloat16)
    c, s = cos_ref[...], sin_ref[...]
    q = jnp.dot(xn, wq_ref[...], preferred_element_type=jnp.float32)
    q_ref[...] = (q * c + _rotate_half_flat(q) * s).astype(jnp.bfloat16)
    k = jnp.dot(xn, wk_ref[...], preferred_element_type=jnp.float32)
    k_ref[...] = (k * c + _rotate_half_flat(k) * s).astype(jnp.bfloat16)
    v_ref[...] = jnp.dot(xn, wv_ref[...],
                         preferred_element_type=jnp.float32).astype(jnp.bfloat16)


def _qkv_rope(x, cosE, sinE, ln1_w, wqT, wkT, wvT):
    row = lambda i: (i, 0)
    full = lambda i: (0, 0)
    return pl.pallas_call(
        _k1_body,
        grid=(S // RB,),
        in_specs=[
            pl.BlockSpec((RB, D), row),
            pl.BlockSpec((RB, D), row),
            pl.BlockSpec((RB, D), row),
            pl.BlockSpec((1, D), full),
            pl.BlockSpec((D, D), full),
            pl.BlockSpec((D, D), full),
            pl.BlockSpec((D, D), full),
        ],
        out_specs=[pl.BlockSpec((RB, D), row)] * 3,
        out_shape=[jax.ShapeDtypeStruct((S, D), jnp.bfloat16)] * 3,
    )(x, cosE, sinE, ln1_w, wqT, wkT, wvT)


def _attn_body(kv_len, qb0, q_ref, k_ref, v_ref, o_ref):
    q = q_ref[0]
    k = k_ref[0]
    s = lax.dot_general(q, k, (((1,), (1,)), ((), ())),
                        preferred_element_type=jnp.float32) * (1.0 / 8.0)
    qb = pl.program_id(1) + qb0
    rows = qb * RB + lax.broadcasted_iota(jnp.int32, (RB, kv_len), 0)
    cols = lax.broadcasted_iota(jnp.int32, (RB, kv_len), 1)
    s = jnp.where(rows >= cols, s, -1e30)
    m = jnp.max(s, axis=-1, keepdims=True)
    p = jnp.exp(s - m)
    p = (p / jnp.sum(p, axis=-1, keepdims=True)).astype(jnp.bfloat16)
    o_ref[0] = lax.dot_general(p, v_ref[0], (((1,), (0,)), ((), ())),
                               preferred_element_type=jnp.float32
                               ).astype(jnp.bfloat16)


def _attention_part(qh, kh, vh, qb0, n_qb, kv_len):
    return pl.pallas_call(
        functools.partial(_attn_body, kv_len, qb0),
        grid=(H, n_qb),
        in_specs=[
            pl.BlockSpec((1, RB, HD), lambda h, qb: (h, qb + qb0, 0)),
            pl.BlockSpec((1, kv_len, HD), lambda h, qb: (h, 0, 0)),
            pl.BlockSpec((1, kv_len, HD), lambda h, qb: (h, 0, 0)),
        ],
        out_specs=pl.BlockSpec((1, RB, HD), lambda h, qb: (h, qb, 0)),
        out_shape=jax.ShapeDtypeStruct((H, n_qb * RB, HD), jnp.bfloat16),
    )(qh, kh, vh)


def _attention(qh, kh, vh):
    nq = S // RB
    step = nq // 4
    parts = [
        _attention_part(qh, kh, vh, i * step, step, (i + 1) * step * RB)
        for i in range(4)
    ]
    return jnp.concatenate(parts, axis=1)


def _k3_body(x_ref, ao_ref, ow_ref, ln2_ref, gw_ref, h1_ref, x2_ref,
             i0_ref, i1_ref, w0_ref, w1_ref):
    proj = jnp.dot(ao_ref[...], ow_ref[...], preferred_element_type=jnp.float32)
    h1 = x_ref[...] + proj
    h1_ref[...] = h1
    var = jnp.mean(h1 * h1, axis=-1, keepdims=True)
    x2 = (h1 * lax.rsqrt(var + EPS)) * ln2_ref[...]
    x2_ref[...] = x2
    lg = jnp.dot(x2, gw_ref[...], preferred_element_type=jnp.float32)
    eidx = lax.broadcasted_iota(jnp.int32, (RB, E), 1)
    m1 = jnp.max(lg, axis=-1, keepdims=True)
    i0 = jnp.min(jnp.where(lg == m1, eidx, E), axis=-1, keepdims=True)
    lg2 = jnp.where(eidx == i0, -jnp.inf, lg)
    m2 = jnp.max(lg2, axis=-1, keepdims=True)
    i1 = jnp.min(jnp.where(lg2 == m2, eidx, E), axis=-1, keepdims=True)
    z = jnp.sum(jnp.exp(lg - m1), axis=-1, keepdims=True)
    i0_ref[...] = i0
    i1_ref[...] = i1
    w0_ref[...] = 1.0 / z
    w1_ref[...] = jnp.exp(m2 - m1) / z


def _oproj_ln2_gate(x, ao, owT, ln2_w, gwT):
    row = lambda i: (i, 0)
    full = lambda i: (0, 0)
    return pl.pallas_call(
        _k3_body,
        grid=(S // RB,),
        in_specs=[
            pl.BlockSpec((RB, D), row),
            pl.BlockSpec((RB, D), row),
            pl.BlockSpec((D, D), full),
            pl.BlockSpec((1, D), full),
            pl.BlockSpec((D, E), full),
        ],
        out_specs=[
            pl.BlockSpec((RB, D), row),
            pl.BlockSpec((RB, D), row),
            pl.BlockSpec((RB, 1), row),
            pl.BlockSpec((RB, 1), row),
            pl.BlockSpec((RB, 1), row),
            pl.BlockSpec((RB, 1), row),
        ],
        out_shape=[
            jax.ShapeDtypeStruct((S, D), jnp.float32),
            jax.ShapeDtypeStruct((S, D), jnp.float32),
            jax.ShapeDtypeStruct((S, 1), jnp.int32),
            jax.ShapeDtypeStruct((S, 1), jnp.int32),
            jax.ShapeDtypeStruct((S, 1), jnp.float32),
            jax.ShapeDtypeStruct((S, 1), jnp.float32),
        ],
    )(x, ao, owT, ln2_w, gwT)


def _silu(a):
    return a * (1.0 / (1.0 + jnp.exp(-a)))


def _moe_body(nlive_ref, be_ref, xg_ref, eg_ref, eu_ref, ed_ref, yg_ref):
    @pl.when(pl.program_id(0) < nlive_ref[0])
    def _():
        xb = xg_ref[...].astype(jnp.bfloat16)
        a = lax.dot_general(xb, eg_ref[0], (((1,), (1,)), ((), ())),
                            preferred_element_type=jnp.float32)
        u = lax.dot_general(xb, eu_ref[0], (((1,), (1,)), ((), ())),
                            preferred_element_type=jnp.float32)
        s = (_silu(a) * u).astype(jnp.bfloat16)
        yg_ref[...] = lax.dot_general(s, ed_ref[0], (((1,), (1,)), ((), ())),
                                      preferred_element_type=jnp.float32)


def _grouped_ffn(xg, egb, eub, edb, be, nlive):
    grid_spec = pltpu.PrefetchScalarGridSpec(
        num_scalar_prefetch=2,
        grid=(NB,),
        in_specs=[
            pl.BlockSpec((BLK, D), lambda b, nl, be: (b, 0)),
            pl.BlockSpec((1, DFF, D), lambda b, nl, be: (be[b], 0, 0)),
            pl.BlockSpec((1, DFF, D), lambda b, nl, be: (be[b], 0, 0)),
            pl.BlockSpec((1, D, DFF), lambda b, nl, be: (be[b], 0, 0)),
        ],
        out_specs=pl.BlockSpec((BLK, D), lambda b, nl, be: (b, 0)),
    )
    return pl.pallas_call(
        _moe_body,
        grid_spec=grid_spec,
        out_shape=jax.ShapeDtypeStruct((NP_PAD, D), jnp.float32),
    )(nlive, be, xg, egb, eub, edb)


def _shared_body(h1_ref, x2_ref, sg_ref, su_ref, sd_ref, o_ref):
    xb = x2_ref[...].astype(jnp.bfloat16)
    a = jnp.dot(xb, sg_ref[...], preferred_element_type=jnp.float32)
    u = jnp.dot(xb, su_ref[...], preferred_element_type=jnp.float32)
    s = (_silu(a) * u).astype(jnp.bfloat16)
    o_ref[...] = h1_ref[...] + jnp.dot(s, sd_ref[...],
                                       preferred_element_type=jnp.float32)


def _shared_ffn(h1, x2, sgT, suT, sdT):
    row = lambda i: (i, 0)
    full = lambda i: (0, 0)
    return pl.pallas_call(
        _shared_body,
        grid=(S // RB,),
        in_specs=[
            pl.BlockSpec((RB, D), row),
            pl.BlockSpec((RB, D), row),
            pl.BlockSpec((D, SFF), full),
            pl.BlockSpec((D, SFF), full),
            pl.BlockSpec((SFF, D), full),
        ],
        out_specs=pl.BlockSpec((RB, D), row),
        out_shape=jax.ShapeDtypeStruct((S, D), jnp.float32),
    )(h1, x2, sgT, suT, sdT)


def _combine_body(base_ref, g0_ref, g1_ref, w0_ref, w1_ref, o_ref):
    o_ref[...] = (base_ref[...] + w0_ref[...] * g0_ref[...]
                  + w1_ref[...] * g1_ref[...])


def _combine(base, gathered, w0, w1):
    row = lambda i: (i, 0)
    nq = S // RB
    return pl.pallas_call(
        _combine_body,
        grid=(nq,),
        in_specs=[
            pl.BlockSpec((RB, D), row),
            pl.BlockSpec((RB, D), row),
            pl.BlockSpec((RB, D), lambda i: (i + nq, 0)),
            pl.BlockSpec((RB, 1), row),
            pl.BlockSpec((RB, 1), row),
        ],
        out_specs=pl.BlockSpec((RB, D), row),
        out_shape=jax.ShapeDtypeStruct((S, D), jnp.float32),
    )(base, gathered, gathered, w0, w1)


def _sc_route(i0, i1):
    mesh = plsc.VectorSubcoreMesh(core_axis_name="c", subcore_axis_name="s")
    L = 16
    n_tok_ch = S // L

    @functools.partial(
        pl.kernel, mesh=mesh,
        out_type=(
            jax.ShapeDtypeStruct((2 * S,), jnp.int32),
            jax.ShapeDtypeStruct((32,), jnp.int32),
            jax.ShapeDtypeStruct((16,), jnp.int32),
        ),
        scratch_types=[
            pltpu.VMEM((S,), jnp.int32),
            pltpu.VMEM((S,), jnp.int32),
            pltpu.VMEM((2 * S,), jnp.int32),
            pltpu.VMEM((32,), jnp.int32),
            pltpu.VMEM((16,), jnp.int32),
        ],
    )
    def k(i0_hbm, i1_hbm, back_hbm, be_hbm, nl_hbm,
          i0_v, i1_v, back_v, be_v, nl_v):
        wid = lax.axis_index("s") * 2 + lax.axis_index("c")

        @pl.when(wid == 0)
        def _():
            pltpu.sync_copy(i0_hbm, i0_v)
            pltpu.sync_copy(i1_hbm, i1_v)
            lane = lax.iota(jnp.int32, L)
            zero = jnp.zeros((L,), jnp.int32)
            last = zero + (L - 1)
            dn = lax.GatherDimensionNumbers(offset_dims=(),
                                            collapsed_slice_dims=(0,),
                                            start_index_map=(0,))

            def perm(v, idx):
                return lax.gather(v, idx[:, None], dimension_numbers=dn,
                                  slice_sizes=(1,),
                                  mode=lax.GatherScatterMode.PROMISE_IN_BOUNDS)

            def incl_scan(v):
                for sh in (1, 2, 4, 8):
                    v = v + jnp.where(lane >= sh,
                                      perm(v, jnp.maximum(lane - sh, 0)), 0)
                return v

            def splat_last(v):
                return perm(v, last)

            def count_body(c, accs):
                e0 = i0_v[pl.ds(c * L, L)]
                e1 = i1_v[pl.ds(c * L, L)]
                return tuple(
                    accs[e]
                    + jnp.where(e0 == e, 1, 0)
                    + jnp.where(e1 == e, 1, 0)
                    for e in range(E))

            accs = lax.fori_loop(0, n_tok_ch, count_body, (zero,) * E)
            cnts = [splat_last(incl_scan(a)) for a in accs]
            pcs = [((c + BLK - 1) >> 8) << 8 for c in cnts]
            pss = [zero]
            for e in range(E):
                pss.append(pss[e] + pcs[e])
            nl_v[...] = pss[E] >> 8
            for c in range(2):
                bpos = (c * L + lane) * BLK
                acc = zero
                for e in range(E):
                    acc = acc + jnp.where(bpos >= pss[e] + pcs[e], 1, 0)
                be_v[pl.ds(c * L, L)] = jnp.minimum(acc, E - 1)

            def place_body(c, offs):
                for kk, ref in ((0, i0_v), (1, i1_v)):
                    e_vec = ref[pl.ds(c * L, L)]
                    dst = zero
                    new_offs = []
                    for e in range(E):
                        mi = jnp.where(e_vec == e, 1, 0)
                        inc = incl_scan(mi)
                        dst = jnp.where(e_vec == e, offs[e] + inc - mi, dst)
                        new_offs.append(offs[e] + splat_last(inc))
                    offs = tuple(new_offs)
                    back_v[pl.ds(kk * S + c * L, L)] = dst
                return offs

            lax.fori_loop(0, n_tok_ch, place_body, tuple(pss[:E]))

            pltpu.sync_copy(back_v, back_hbm)
            pltpu.sync_copy(be_v, be_hbm)
            pltpu.sync_copy(nl_v, nl_hbm)

    return k(i0, i1)


def _sc_scatter_rows(table, idx, out_rows):
    info = plsc.get_sparse_core_info()
    nw = info.num_cores * info.num_subcores
    n_tok, d = table.shape
    per_w = n_tok // nw
    mesh = plsc.VectorSubcoreMesh(core_axis_name="c", subcore_axis_name="s")

    @functools.partial(
        pl.kernel, mesh=mesh,
        out_type=jax.ShapeDtypeStruct((out_rows, d), table.dtype),
        scratch_types=[
            pltpu.VMEM((per_w,), jnp.int32),
            pltpu.VMEM((per_w,), jnp.int32),
            pltpu.VMEM((per_w, d), table.dtype),
            pltpu.SemaphoreType.DMA,
            pltpu.SemaphoreType.DMA,
        ],
    )
    def k(table_hbm, idx_hbm, out_hbm, idx0_v, idx1_v, buf, sem0, sem1):
        wid = lax.axis_index("s") * info.num_cores + lax.axis_index("c")
        tbase = wid * per_w
        pltpu.sync_copy(idx_hbm.at[pl.ds(tbase, per_w)], idx0_v)
        pltpu.sync_copy(idx_hbm.at[pl.ds(S + tbase, per_w)], idx1_v)
        pltpu.sync_copy(table_hbm.at[pl.ds(tbase, per_w)], buf)
        h0 = pltpu.async_copy(buf, out_hbm.at[idx0_v], sem0)
        h1 = pltpu.async_copy(buf, out_hbm.at[idx1_v], sem1)
        h0.wait()
        h1.wait()

    return k(table, idx)


def _sc_gather_rows(table, idx, chunk=32, nbuf=3):
    info = plsc.get_sparse_core_info()
    nw = info.num_cores * info.num_subcores
    n, tail = idx.shape[0], table.shape[1:]
    per_w = n // nw
    n_ch = per_w // chunk
    assert n_ch * chunk == per_w
    mesh = plsc.VectorSubcoreMesh(core_axis_name="c", subcore_axis_name="s")

    @functools.partial(
        pl.kernel, mesh=mesh,
        out_type=jax.ShapeDtypeStruct((n,) + tail, table.dtype),
        scratch_types=(
            [pltpu.VMEM((per_w,), jnp.int32)]
            + [pltpu.VMEM((chunk,) + tail, table.dtype)] * nbuf
            + [pltpu.SemaphoreType.DMA] * (2 * nbuf)
        ),
    )
    def k(table_hbm, idx_hbm, out_hbm, idx_v, *bufs_sems):
        bufs = bufs_sems[:nbuf]
        gsems = bufs_sems[nbuf:2 * nbuf]
        wsems = bufs_sems[2 * nbuf:]
        wid = lax.axis_index("s") * info.num_cores + lax.axis_index("c")
        base = wid * per_w
        pltpu.sync_copy(idx_hbm.at[pl.ds(base, per_w)], idx_v)

        def start_gather(c, b):
            return pltpu.async_copy(
                table_hbm.at[idx_v.at[pl.ds(c * chunk, chunk)]],
                bufs[b], gsems[b])

        gh, wh = {}, {}
        for c in range(min(nbuf, n_ch)):
            gh[c] = start_gather(c, c % nbuf)
        for c in range(n_ch):
            b = c % nbuf
            gh[c].wait()
            wh[c] = pltpu.async_copy(
                bufs[b], out_hbm.at[pl.ds(base + c * chunk, chunk)], wsems[b])
            if c + nbuf < n_ch:
                wh[c].wait()
                gh[c + nbuf] = start_gather(c + nbuf, b)
        for c in range(max(0, n_ch - nbuf), n_ch):
            wh[c].wait()

    return k(table, idx)


def kernel(hidden_states, position_ids, ln1_w, q_w, k_w, v_w, o_w, ln2_w,
           gate_w, eg, eu, ed, sg, su, sd):
    x = hidden_states.reshape(S, D)

    inv_freq = 1.0 / (ROPE_BASE ** (jnp.arange(0, HD, 2, dtype=jnp.float32) / HD))
    freqs = jnp.outer(jnp.arange(S, dtype=jnp.float32), inv_freq)
    emb = jnp.concatenate([freqs, freqs], axis=-1)
    pos = position_ids.reshape(S)
    cosE = jnp.tile(jnp.cos(emb)[pos], (1, H))
    sinE = jnp.tile(jnp.sin(emb)[pos], (1, H))

    wqT = q_w.T.astype(jnp.bfloat16)
    wkT = k_w.T.astype(jnp.bfloat16)
    wvT = v_w.T.astype(jnp.bfloat16)
    q, k, v = _qkv_rope(x, cosE, sinE, ln1_w.reshape(1, D), wqT, wkT, wvT)

    qh = q.reshape(S, H, HD).transpose(1, 0, 2)
    kh = k.reshape(S, H, HD).transpose(1, 0, 2)
    vh = v.reshape(S, H, HD).transpose(1, 0, 2)
    ao = _attention(qh, kh, vh).transpose(1, 0, 2).reshape(S, D)

    h1, x2, i0, i1, w0, w1 = _oproj_ln2_gate(x, ao,
                                             o_w.T.astype(jnp.bfloat16),
                                             ln2_w.reshape(1, D), gate_w.T)

    back_idx, be, nlive = _sc_route(i0.reshape(S), i1.reshape(S))
    xg = _sc_scatter_rows(x2, back_idx, NP_PAD)
    yg = _grouped_ffn(xg, eg.astype(jnp.bfloat16), eu.astype(jnp.bfloat16),
                      ed.astype(jnp.bfloat16), be, nlive)
    gathered = _sc_gather_rows(yg, back_idx)

    base = _shared_ffn(h1, x2, sg.T.astype(jnp.bfloat16),
                       su.T.astype(jnp.bfloat16), sd.T.astype(jnp.bfloat16))
    out = _combine(base, gathered, w0, w1)
    return out.reshape(1, S, D)


# speedup vs baseline: 1.3836x; 1.0131x over previous
import functools

import jax
import jax.numpy as jnp
import numpy as np
from jax import lax
from jax.experimental import pallas as pl
from jax.experimental.pallas import tpu as pltpu
from jax.experimental.pallas import tpu_sc as plsc

S = 2048
D = 1024
H = 16
HD = 64
E = 8
DFF = 1408
SFF = 2816
EPS = 1e-6
ROPE_BASE = 10000.0

RB = 256
BLK = 256
NP_PAD = 4096 + 8 * (BLK - 1)
NP_PAD = ((NP_PAD + BLK - 1) // BLK) * BLK
NB = NP_PAD // BLK

def _rotate_half_flat(x):
    first = (lax.broadcasted_iota(jnp.int32, x.shape, 1) & 63) < 32
    return jnp.where(first, -jnp.roll(x, -32, axis=1), jnp.roll(x, 32, axis=1))


def _k1_body(x_ref, cos_ref, sin_ref, ln1_ref, wq_ref, wk_ref,
             wv_ref, q_ref, k_ref, v_ref):
    x = x_ref[...]
    var = jnp.mean(x * x, axis=-1, keepdims=True)
    xn = ((x * lax.rsqrt(var + EPS)) * ln1_ref[...]).astype(jnp.bfloat16)
    c, s = cos_ref[...], sin_ref[...]
    q = jnp.dot(xn, wq_ref[...], preferred_element_type=jnp.float32)
    q_ref[...] = (q * c + _rotate_half_flat(q) * s).astype(jnp.bfloat16)
    k = jnp.dot(xn, wk_ref[...], preferred_element_type=jnp.float32)
    k_ref[...] = (k * c + _rotate_half_flat(k) * s).astype(jnp.bfloat16)
    v_ref[...] = jnp.dot(xn, wv_ref[...],
                         preferred_element_type=jnp.float32).astype(jnp.bfloat16)


def _qkv_rope(x, cosE, sinE, ln1_w, wqT, wkT, wvT):
    row = lambda i: (i, 0)
    full = lambda i: (0, 0)
    return pl.pallas_call(
        _k1_body,
        grid=(S // RB,),
        in_specs=[
            pl.BlockSpec((RB, D), row),
            pl.BlockSpec((RB, D), row),
            pl.BlockSpec((RB, D), row),
            pl.BlockSpec((1, D), full),
            pl.BlockSpec((D, D), full),
            pl.BlockSpec((D, D), full),
            pl.BlockSpec((D, D), full),
        ],
        out_specs=[pl.BlockSpec((RB, D), row)] * 3,
        out_shape=[jax.ShapeDtypeStruct((S, D), jnp.bfloat16)] * 3,
    )(x, cosE, sinE, ln1_w, wqT, wkT, wvT)


def _attn_body(kv_len, qb0, q_ref, k_ref, v_ref, o_ref):
    q = q_ref[0]
    k = k_ref[0]
    s = lax.dot_general(q, k, (((1,), (1,)), ((), ())),
                        preferred_element_type=jnp.float32) * (1.0 / 8.0)
    qb = pl.program_id(1) + qb0
    rows = qb * RB + lax.broadcasted_iota(jnp.int32, (RB, kv_len), 0)
    cols = lax.broadcasted_iota(jnp.int32, (RB, kv_len), 1)
    s = jnp.where(rows >= cols, s, -1e30)
    m = jnp.max(s, axis=-1, keepdims=True)
    p = jnp.exp(s - m)
    p = (p / jnp.sum(p, axis=-1, keepdims=True)).astype(jnp.bfloat16)
    o_ref[0] = lax.dot_general(p, v_ref[0], (((1,), (0,)), ((), ())),
                               preferred_element_type=jnp.float32
                               ).astype(jnp.bfloat16)


def _attention_part(qh, kh, vh, qb0, n_qb, kv_len):
    return pl.pallas_call(
        functools.partial(_attn_body, kv_len, qb0),
        grid=(H, n_qb),
        in_specs=[
            pl.BlockSpec((1, RB, HD), lambda h, qb: (h, qb + qb0, 0)),
            pl.BlockSpec((1, kv_len, HD), lambda h, qb: (h, 0, 0)),
            pl.BlockSpec((1, kv_len, HD), lambda h, qb: (h, 0, 0)),
        ],
        out_specs=pl.BlockSpec((1, RB, HD), lambda h, qb: (h, qb, 0)),
        out_shape=jax.ShapeDtypeStruct((H, n_qb * RB, HD), jnp.bfloat16),
    )(qh, kh, vh)


def _attention(qh, kh, vh):
    nq = S // RB
    step = nq // 4
    parts = [
        _attention_part(qh, kh, vh, i * step, step, (i + 1) * step * RB)
        for i in range(4)
    ]
    return jnp.concatenate(parts, axis=1)


def _k3_body(x_ref, ao_ref, ow_ref, ln2_ref, gw_ref, h1_ref, x2_ref,
             i0_ref, i1_ref, w0_ref, w1_ref):
    proj = jnp.dot(ao_ref[...], ow_ref[...], preferred_element_type=jnp.float32)
    h1 = x_ref[...] + proj
    h1_ref[...] = h1
    var = jnp.mean(h1 * h1, axis=-1, keepdims=True)
    x2 = (h1 * lax.rsqrt(var + EPS)) * ln2_ref[...]
    x2_ref[...] = x2
    lg = jnp.dot(x2, gw_ref[...], preferred_element_type=jnp.float32)
    eidx = lax.broadcasted_iota(jnp.int32, (RB, E), 1)
    m1 = jnp.max(lg, axis=-1, keepdims=True)
    i0 = jnp.min(jnp.where(lg == m1, eidx, E), axis=-1, keepdims=True)
    lg2 = jnp.where(eidx == i0, -jnp.inf, lg)
    m2 = jnp.max(lg2, axis=-1, keepdims=True)
    i1 = jnp.min(jnp.where(lg2 == m2, eidx, E), axis=-1, keepdims=True)
    z = jnp.sum(jnp.exp(lg - m1), axis=-1, keepdims=True)
    i0_ref[...] = i0
    i1_ref[...] = i1
    w0_ref[...] = 1.0 / z
    w1_ref[...] = jnp.exp(m2 - m1) / z


def _oproj_ln2_gate(x, ao, owT, ln2_w, gwT):
    row = lambda i: (i, 0)
    full = lambda i: (0, 0)
    return pl.pallas_call(
        _k3_body,
        grid=(S // RB,),
        in_specs=[
            pl.BlockSpec((RB, D), row),
            pl.BlockSpec((RB, D), row),
            pl.BlockSpec((D, D), full),
            pl.BlockSpec((1, D), full),
            pl.BlockSpec((D, E), full),
        ],
        out_specs=[
            pl.BlockSpec((RB, D), row),
            pl.BlockSpec((RB, D), row),
            pl.BlockSpec((RB, 1), row),
            pl.BlockSpec((RB, 1), row),
            pl.BlockSpec((RB, 1), row),
            pl.BlockSpec((RB, 1), row),
        ],
        out_shape=[
            jax.ShapeDtypeStruct((S, D), jnp.float32),
            jax.ShapeDtypeStruct((S, D), jnp.float32),
            jax.ShapeDtypeStruct((S, 1), jnp.int32),
            jax.ShapeDtypeStruct((S, 1), jnp.int32),
            jax.ShapeDtypeStruct((S, 1), jnp.float32),
            jax.ShapeDtypeStruct((S, 1), jnp.float32),
        ],
    )(x, ao, owT, ln2_w, gwT)


def _silu(a):
    return a * (1.0 / (1.0 + jnp.exp(-a)))


def _moe_body(nlive_ref, be_ref, xg_ref, eg_ref, eu_ref, ed_ref, yg_ref):
    @pl.when(pl.program_id(0) < nlive_ref[0])
    def _():
        xb = xg_ref[...].astype(jnp.bfloat16)
        a = lax.dot_general(xb, eg_ref[0], (((1,), (1,)), ((), ())),
                            preferred_element_type=jnp.float32)
        u = lax.dot_general(xb, eu_ref[0], (((1,), (1,)), ((), ())),
                            preferred_element_type=jnp.float32)
        s = (_silu(a) * u).astype(jnp.bfloat16)
        yg_ref[...] = lax.dot_general(s, ed_ref[0], (((1,), (1,)), ((), ())),
                                      preferred_element_type=jnp.float32)


def _grouped_ffn(xg, egb, eub, edb, be, nlive):
    grid_spec = pltpu.PrefetchScalarGridSpec(
        num_scalar_prefetch=2,
        grid=(NB,),
        in_specs=[
            pl.BlockSpec((BLK, D), lambda b, nl, be: (b, 0)),
            pl.BlockSpec((1, DFF, D), lambda b, nl, be: (be[b], 0, 0)),
            pl.BlockSpec((1, DFF, D), lambda b, nl, be: (be[b], 0, 0)),
            pl.BlockSpec((1, D, DFF), lambda b, nl, be: (be[b], 0, 0)),
        ],
        out_specs=pl.BlockSpec((BLK, D), lambda b, nl, be: (b, 0)),
    )
    return pl.pallas_call(
        _moe_body,
        grid_spec=grid_spec,
        out_shape=jax.ShapeDtypeStruct((NP_PAD, D), jnp.float32),
    )(nlive, be, xg, egb, eub, edb)


def _shared_body(h1_ref, x2_ref, sg_ref, su_ref, sd_ref, g0_ref, g1_ref,
                 w0_ref, w1_ref, o_ref):
    xb = x2_ref[...].astype(jnp.bfloat16)
    a = jnp.dot(xb, sg_ref[...], preferred_element_type=jnp.float32)
    u = jnp.dot(xb, su_ref[...], preferred_element_type=jnp.float32)
    s = (_silu(a) * u).astype(jnp.bfloat16)
    o_ref[...] = (h1_ref[...]
                  + jnp.dot(s, sd_ref[...], preferred_element_type=jnp.float32)
                  + w0_ref[...] * g0_ref[...] + w1_ref[...] * g1_ref[...])


def _shared_ffn(h1, x2, sgT, suT, sdT, gathered, w0, w1):
    row = lambda i: (i, 0)
    full = lambda i: (0, 0)
    nq = S // RB
    return pl.pallas_call(
        _shared_body,
        grid=(nq,),
        in_specs=[
            pl.BlockSpec((RB, D), row),
            pl.BlockSpec((RB, D), row),
            pl.BlockSpec((D, SFF), full),
            pl.BlockSpec((D, SFF), full),
            pl.BlockSpec((SFF, D), full),
            pl.BlockSpec((RB, D), row),
            pl.BlockSpec((RB, D), lambda i: (i + nq, 0)),
            pl.BlockSpec((RB, 1), row),
            pl.BlockSpec((RB, 1), row),
        ],
        out_specs=pl.BlockSpec((RB, D), row),
        out_shape=jax.ShapeDtypeStruct((S, D), jnp.float32),
    )(h1, x2, sgT, suT, sdT, gathered, gathered, w0, w1)


def _sc_route(i0, i1):
    mesh = plsc.VectorSubcoreMesh(core_axis_name="c", subcore_axis_name="s")
    L = 16
    n_tok_ch = S // L

    @functools.partial(
        pl.kernel, mesh=mesh,
        out_type=(
            jax.ShapeDtypeStruct((2 * S,), jnp.int32),
            jax.ShapeDtypeStruct((32,), jnp.int32),
            jax.ShapeDtypeStruct((16,), jnp.int32),
        ),
        scratch_types=[
            pltpu.VMEM((S,), jnp.int32),
            pltpu.VMEM((S,), jnp.int32),
            pltpu.VMEM((2 * S,), jnp.int32),
            pltpu.VMEM((32,), jnp.int32),
            pltpu.VMEM((16,), jnp.int32),
        ],
    )
    def k(i0_hbm, i1_hbm, back_hbm, be_hbm, nl_hbm,
          i0_v, i1_v, back_v, be_v, nl_v):
        wid = lax.axis_index("s") * 2 + lax.axis_index("c")

        @pl.when(wid == 0)
        def _():
            pltpu.sync_copy(i0_hbm, i0_v)
            pltpu.sync_copy(i1_hbm, i1_v)
            lane = lax.iota(jnp.int32, L)
            zero = jnp.zeros((L,), jnp.int32)
            last = zero + (L - 1)
            dn = lax.GatherDimensionNumbers(offset_dims=(),
                                            collapsed_slice_dims=(0,),
                                            start_index_map=(0,))

            def perm(v, idx):
                return lax.gather(v, idx[:, None], dimension_numbers=dn,
                                  slice_sizes=(1,),
                                  mode=lax.GatherScatterMode.PROMISE_IN_BOUNDS)

            def incl_scan(v):
                for sh in (1, 2, 4, 8):
                    v = v + jnp.where(lane >= sh,
                                      perm(v, jnp.maximum(lane - sh, 0)), 0)
                return v

            def splat_last(v):
                return perm(v, last)

            def count_body(c, accs):
                e0 = i0_v[pl.ds(c * L, L)]
                e1 = i1_v[pl.ds(c * L, L)]
                return tuple(
                    accs[e]
                    + jnp.where(e0 == e, 1, 0)
                    + jnp.where(e1 == e, 1, 0)
                    for e in range(E))

            accs = lax.fori_loop(0, n_tok_ch, count_body, (zero,) * E)
            cnts = [splat_last(incl_scan(a)) for a in accs]
            pcs = [((c + BLK - 1) >> 8) << 8 for c in cnts]
            pss = [zero]
            for e in range(E):
                pss.append(pss[e] + pcs[e])
            nl_v[...] = pss[E] >> 8
            for c in range(2):
                bpos = (c * L + lane) * BLK
                acc = zero
                for e in range(E):
                    acc = acc + jnp.where(bpos >= pss[e] + pcs[e], 1, 0)
                be_v[pl.ds(c * L, L)] = jnp.minimum(acc, E - 1)

            def place_body(c, offs):
                for kk, ref in ((0, i0_v), (1, i1_v)):
                    e_vec = ref[pl.ds(c * L, L)]
                    dst = zero
                    new_offs = []
                    for e in range(E):
                        mi = jnp.where(e_vec == e, 1, 0)
                        inc = incl_scan(mi)
                        dst = jnp.where(e_vec == e, offs[e] + inc - mi, dst)
                        new_offs.append(offs[e] + splat_last(inc))
                    offs = tuple(new_offs)
                    back_v[pl.ds(kk * S + c * L, L)] = dst
                return offs

            lax.fori_loop(0, n_tok_ch, place_body, tuple(pss[:E]))

            pltpu.sync_copy(back_v, back_hbm)
            pltpu.sync_copy(be_v, be_hbm)
            pltpu.sync_copy(nl_v, nl_hbm)

    return k(i0, i1)


def _sc_scatter_rows(table, idx, out_rows):
    info = plsc.get_sparse_core_info()
    nw = info.num_cores * info.num_subcores
    n_tok, d = table.shape
    per_w = n_tok // nw
    mesh = plsc.VectorSubcoreMesh(core_axis_name="c", subcore_axis_name="s")

    @functools.partial(
        pl.kernel, mesh=mesh,
        out_type=jax.ShapeDtypeStruct((out_rows, d), table.dtype),
        scratch_types=[
            pltpu.VMEM((per_w,), jnp.int32),
            pltpu.VMEM((per_w,), jnp.int32),
            pltpu.VMEM((per_w, d), table.dtype),
            pltpu.SemaphoreType.DMA,
            pltpu.SemaphoreType.DMA,
        ],
    )
    def k(table_hbm, idx_hbm, out_hbm, idx0_v, idx1_v, buf, sem0, sem1):
        wid = lax.axis_index("s") * info.num_cores + lax.axis_index("c")
        tbase = wid * per_w
        pltpu.sync_copy(idx_hbm.at[pl.ds(tbase, per_w)], idx0_v)
        pltpu.sync_copy(idx_hbm.at[pl.ds(S + tbase, per_w)], idx1_v)
        pltpu.sync_copy(table_hbm.at[pl.ds(tbase, per_w)], buf)
        h0 = pltpu.async_copy(buf, out_hbm.at[idx0_v], sem0)
        h1 = pltpu.async_copy(buf, out_hbm.at[idx1_v], sem1)
        h0.wait()
        h1.wait()

    return k(table, idx)


def _sc_gather_rows(table, idx, chunk=32, nbuf=3):
    info = plsc.get_sparse_core_info()
    nw = info.num_cores * info.num_subcores
    n, tail = idx.shape[0], table.shape[1:]
    per_w = n // nw
    n_ch = per_w // chunk
    assert n_ch * chunk == per_w
    mesh = plsc.VectorSubcoreMesh(core_axis_name="c", subcore_axis_name="s")

    @functools.partial(
        pl.kernel, mesh=mesh,
        out_type=jax.ShapeDtypeStruct((n,) + tail, table.dtype),
        scratch_types=(
            [pltpu.VMEM((per_w,), jnp.int32)]
            + [pltpu.VMEM((chunk,) + tail, table.dtype)] * nbuf
            + [pltpu.SemaphoreType.DMA] * (2 * nbuf)
        ),
    )
    def k(table_hbm, idx_hbm, out_hbm, idx_v, *bufs_sems):
        bufs = bufs_sems[:nbuf]
        gsems = bufs_sems[nbuf:2 * nbuf]
        wsems = bufs_sems[2 * nbuf:]
        wid = lax.axis_index("s") * info.num_cores + lax.axis_index("c")
        base = wid * per_w
        pltpu.sync_copy(idx_hbm.at[pl.ds(base, per_w)], idx_v)

        def start_gather(c, b):
            return pltpu.async_copy(
                table_hbm.at[idx_v.at[pl.ds(c * chunk, chunk)]],
                bufs[b], gsems[b])

        gh, wh = {}, {}
        for c in range(min(nbuf, n_ch)):
            gh[c] = start_gather(c, c % nbuf)
        for c in range(n_ch):
            b = c % nbuf
            gh[c].wait()
            wh[c] = pltpu.async_copy(
                bufs[b], out_hbm.at[pl.ds(base + c * chunk, chunk)], wsems[b])
            if c + nbuf < n_ch:
                wh[c].wait()
                gh[c + nbuf] = start_gather(c + nbuf, b)
        for c in range(max(0, n_ch - nbuf), n_ch):
            wh[c].wait()

    return k(table, idx)


def kernel(hidden_states, position_ids, ln1_w, q_w, k_w, v_w, o_w, ln2_w,
           gate_w, eg, eu, ed, sg, su, sd):
    x = hidden_states.reshape(S, D)

    inv_freq = 1.0 / (ROPE_BASE ** (jnp.arange(0, HD, 2, dtype=jnp.float32) / HD))
    freqs = jnp.outer(jnp.arange(S, dtype=jnp.float32), inv_freq)
    emb = jnp.concatenate([freqs, freqs], axis=-1)
    pos = position_ids.reshape(S)
    cosE = jnp.tile(jnp.cos(emb)[pos], (1, H))
    sinE = jnp.tile(jnp.sin(emb)[pos], (1, H))

    wqT = q_w.T.astype(jnp.bfloat16)
    wkT = k_w.T.astype(jnp.bfloat16)
    wvT = v_w.T.astype(jnp.bfloat16)
    q, k, v = _qkv_rope(x, cosE, sinE, ln1_w.reshape(1, D), wqT, wkT, wvT)

    qh = q.reshape(S, H, HD).transpose(1, 0, 2)
    kh = k.reshape(S, H, HD).transpose(1, 0, 2)
    vh = v.reshape(S, H, HD).transpose(1, 0, 2)
    ao = _attention(qh, kh, vh).transpose(1, 0, 2).reshape(S, D)

    h1, x2, i0, i1, w0, w1 = _oproj_ln2_gate(x, ao,
                                             o_w.T.astype(jnp.bfloat16),
                                             ln2_w.reshape(1, D), gate_w.T)

    back_idx, be, nlive = _sc_route(i0.reshape(S), i1.reshape(S))
    xg = _sc_scatter_rows(x2, back_idx, NP_PAD)
    yg = _grouped_ffn(xg, eg.astype(jnp.bfloat16), eu.astype(jnp.bfloat16),
                      ed.astype(jnp.bfloat16), be, nlive)
    gathered = _sc_gather_rows(yg, back_idx)

    out = _shared_ffn(h1, x2, sg.T.astype(jnp.bfloat16),
                      su.T.astype(jnp.bfloat16), sd.T.astype(jnp.bfloat16),
                      gathered, w0, w1)
    return out.reshape(1, S, D)


# speedup vs baseline: 1.4697x; 1.0622x over previous
import functools

import jax
import jax.numpy as jnp
import numpy as np
from jax import lax
from jax.experimental import pallas as pl
from jax.experimental.pallas import tpu as pltpu
from jax.experimental.pallas import tpu_sc as plsc

S = 2048
D = 1024
H = 16
HD = 64
E = 8
DFF = 1408
SFF = 2816
EPS = 1e-6
ROPE_BASE = 10000.0

RB = 256
BLK = 256
NP_PAD = 4096 + 8 * (BLK - 1)
NP_PAD = ((NP_PAD + BLK - 1) // BLK) * BLK
NB = NP_PAD // BLK

def _rotate_half_flat(x):
    first = (lax.broadcasted_iota(jnp.int32, x.shape, 1) & 63) < 32
    return jnp.where(first, -jnp.roll(x, -32, axis=1), jnp.roll(x, 32, axis=1))


def _k1_body(x_ref, cos_ref, sin_ref, ln1_ref, wq_ref, wk_ref,
             wv_ref, q_ref, k_ref, v_ref):
    x = x_ref[...]
    var = jnp.mean(x * x, axis=-1, keepdims=True)
    xn = ((x * lax.rsqrt(var + EPS)) * ln1_ref[...]).astype(jnp.bfloat16)
    c, s = cos_ref[...], sin_ref[...]
    q = jnp.dot(xn, wq_ref[...], preferred_element_type=jnp.float32)
    q_ref[...] = (q * c + _rotate_half_flat(q) * s).astype(jnp.bfloat16)
    k = jnp.dot(xn, wk_ref[...], preferred_element_type=jnp.float32)
    k_ref[...] = (k * c + _rotate_half_flat(k) * s).astype(jnp.bfloat16)
    v_ref[...] = jnp.dot(xn, wv_ref[...],
                         preferred_element_type=jnp.float32).astype(jnp.bfloat16)


def _qkv_rope(x, cosE, sinE, ln1_w, wqT, wkT, wvT):
    row = lambda i: (i, 0)
    full = lambda i: (0, 0)
    return pl.pallas_call(
        _k1_body,
        grid=(S // RB,),
        in_specs=[
            pl.BlockSpec((RB, D), row),
            pl.BlockSpec((RB, D), row),
            pl.BlockSpec((RB, D), row),
            pl.BlockSpec((1, D), full),
            pl.BlockSpec((D, D), full),
            pl.BlockSpec((D, D), full),
            pl.BlockSpec((D, D), full),
        ],
        out_specs=[pl.BlockSpec((RB, D), row)] * 3,
        out_shape=[jax.ShapeDtypeStruct((S, D), jnp.bfloat16)] * 3,
    )(x, cosE, sinE, ln1_w, wqT, wkT, wvT)


def _attn_body(kv_len, qb0, q_ref, k_ref, v_ref, o_ref):
    q = q_ref[0]
    k = k_ref[0]
    s = lax.dot_general(q, k, (((1,), (1,)), ((), ())),
                        preferred_element_type=jnp.float32) * (1.0 / 8.0)
    qb = pl.program_id(1) + qb0
    rows = qb * RB + lax.broadcasted_iota(jnp.int32, (RB, kv_len), 0)
    cols = lax.broadcasted_iota(jnp.int32, (RB, kv_len), 1)
    s = jnp.where(rows >= cols, s, -1e30)
    m = jnp.max(s, axis=-1, keepdims=True)
    p = jnp.exp(s - m)
    p = (p / jnp.sum(p, axis=-1, keepdims=True)).astype(jnp.bfloat16)
    o_ref[0] = lax.dot_general(p, v_ref[0], (((1,), (0,)), ((), ())),
                               preferred_element_type=jnp.float32
                               ).astype(jnp.bfloat16)


def _attention_part(qh, kh, vh, qb0, n_qb, kv_len):
    return pl.pallas_call(
        functools.partial(_attn_body, kv_len, qb0),
        grid=(H, n_qb),
        in_specs=[
            pl.BlockSpec((1, RB, HD), lambda h, qb: (h, qb + qb0, 0)),
            pl.BlockSpec((1, kv_len, HD), lambda h, qb: (h, 0, 0)),
            pl.BlockSpec((1, kv_len, HD), lambda h, qb: (h, 0, 0)),
        ],
        out_specs=pl.BlockSpec((1, RB, HD), lambda h, qb: (h, qb, 0)),
        out_shape=jax.ShapeDtypeStruct((H, n_qb * RB, HD), jnp.bfloat16),
    )(qh, kh, vh)


def _attention(qh, kh, vh):
    nq = S // RB
    step = nq // 4
    parts = [
        _attention_part(qh, kh, vh, i * step, step, (i + 1) * step * RB)
        for i in range(4)
    ]
    return jnp.concatenate(parts, axis=1)


def _k3_body(x_ref, ao_ref, ow_ref, ln2_ref, gw_ref, h1_ref, x2_ref,
             i0_ref, i1_ref, w0_ref, w1_ref):
    proj = jnp.dot(ao_ref[...], ow_ref[...], preferred_element_type=jnp.float32)
    h1 = x_ref[...] + proj
    h1_ref[...] = h1
    var = jnp.mean(h1 * h1, axis=-1, keepdims=True)
    x2 = (h1 * lax.rsqrt(var + EPS)) * ln2_ref[...]
    x2_ref[...] = x2
    lg = jnp.dot(x2, gw_ref[...], preferred_element_type=jnp.float32)
    eidx = lax.broadcasted_iota(jnp.int32, (RB, E), 1)
    m1 = jnp.max(lg, axis=-1, keepdims=True)
    i0 = jnp.min(jnp.where(lg == m1, eidx, E), axis=-1, keepdims=True)
    lg2 = jnp.where(eidx == i0, -jnp.inf, lg)
    m2 = jnp.max(lg2, axis=-1, keepdims=True)
    i1 = jnp.min(jnp.where(lg2 == m2, eidx, E), axis=-1, keepdims=True)
    z = jnp.sum(jnp.exp(lg - m1), axis=-1, keepdims=True)
    i0_ref[...] = i0
    i1_ref[...] = i1
    w0_ref[...] = 1.0 / z
    w1_ref[...] = jnp.exp(m2 - m1) / z


def _oproj_ln2_gate(x, ao, owT, ln2_w, gwT):
    row = lambda i: (i, 0)
    full = lambda i: (0, 0)
    return pl.pallas_call(
        _k3_body,
        grid=(S // RB,),
        in_specs=[
            pl.BlockSpec((RB, D), row),
            pl.BlockSpec((RB, D), row),
            pl.BlockSpec((D, D), full),
            pl.BlockSpec((1, D), full),
            pl.BlockSpec((D, E), full),
        ],
        out_specs=[
            pl.BlockSpec((RB, D), row),
            pl.BlockSpec((RB, D), row),
            pl.BlockSpec((RB, 1), row),
            pl.BlockSpec((RB, 1), row),
            pl.BlockSpec((RB, 1), row),
            pl.BlockSpec((RB, 1), row),
        ],
        out_shape=[
            jax.ShapeDtypeStruct((S, D), jnp.float32),
            jax.ShapeDtypeStruct((S, D), jnp.float32),
            jax.ShapeDtypeStruct((S, 1), jnp.int32),
            jax.ShapeDtypeStruct((S, 1), jnp.int32),
            jax.ShapeDtypeStruct((S, 1), jnp.float32),
            jax.ShapeDtypeStruct((S, 1), jnp.float32),
        ],
    )(x, ao, owT, ln2_w, gwT)


def _silu(a):
    return a * (1.0 / (1.0 + jnp.exp(-a)))


def _moe_body(nlive_ref, be_ref, xg_ref, eg_ref, eu_ref, ed_ref, yg_ref,
              egb_ref, eub_ref, edb_ref):
    b = pl.program_id(0)

    @pl.when(b < nlive_ref[0])
    def _():
        prev = be_ref[jnp.maximum(b - 1, 0)]
        changed = jnp.logical_or(b == 0, be_ref[b] != prev)

        @pl.when(changed)
        def _():
            egb_ref[...] = eg_ref[0].astype(jnp.bfloat16)
            eub_ref[...] = eu_ref[0].astype(jnp.bfloat16)
            edb_ref[...] = ed_ref[0].astype(jnp.bfloat16)

        xb = xg_ref[...].astype(jnp.bfloat16)
        a = lax.dot_general(xb, egb_ref[...], (((1,), (1,)), ((), ())),
                            preferred_element_type=jnp.float32)
        u = lax.dot_general(xb, eub_ref[...], (((1,), (1,)), ((), ())),
                            preferred_element_type=jnp.float32)
        s = (_silu(a) * u).astype(jnp.bfloat16)
        yg_ref[...] = lax.dot_general(s, edb_ref[...], (((1,), (1,)), ((), ())),
                                      preferred_element_type=jnp.float32)


def _grouped_ffn(xg, egb, eub, edb, be, nlive):
    grid_spec = pltpu.PrefetchScalarGridSpec(
        num_scalar_prefetch=2,
        grid=(NB,),
        in_specs=[
            pl.BlockSpec((BLK, D), lambda b, nl, be: (b, 0)),
            pl.BlockSpec((1, DFF, D), lambda b, nl, be: (be[b], 0, 0)),
            pl.BlockSpec((1, DFF, D), lambda b, nl, be: (be[b], 0, 0)),
            pl.BlockSpec((1, D, DFF), lambda b, nl, be: (be[b], 0, 0)),
        ],
        out_specs=pl.BlockSpec((BLK, D), lambda b, nl, be: (b, 0)),
        scratch_shapes=[
            pltpu.VMEM((DFF, D), jnp.bfloat16),
            pltpu.VMEM((DFF, D), jnp.bfloat16),
            pltpu.VMEM((D, DFF), jnp.bfloat16),
        ],
    )
    return pl.pallas_call(
        _moe_body,
        grid_spec=grid_spec,
        out_shape=jax.ShapeDtypeStruct((NP_PAD, D), jnp.float32),
    )(nlive, be, xg, egb, eub, edb)


def _shared_body(h1_ref, x2_ref, sg_ref, su_ref, sd_ref, g0_ref, g1_ref,
                 w0_ref, w1_ref, o_ref):
    xb = x2_ref[...].astype(jnp.bfloat16)
    a = jnp.dot(xb, sg_ref[...], preferred_element_type=jnp.float32)
    u = jnp.dot(xb, su_ref[...], preferred_element_type=jnp.float32)
    s = (_silu(a) * u).astype(jnp.bfloat16)
    o_ref[...] = (h1_ref[...]
                  + jnp.dot(s, sd_ref[...], preferred_element_type=jnp.float32)
                  + w0_ref[...] * g0_ref[...] + w1_ref[...] * g1_ref[...])


def _shared_ffn(h1, x2, sgT, suT, sdT, gathered, w0, w1):
    row = lambda i: (i, 0)
    full = lambda i: (0, 0)
    nq = S // RB
    return pl.pallas_call(
        _shared_body,
        grid=(nq,),
        in_specs=[
            pl.BlockSpec((RB, D), row),
            pl.BlockSpec((RB, D), row),
            pl.BlockSpec((D, SFF), full),
            pl.BlockSpec((D, SFF), full),
            pl.BlockSpec((SFF, D), full),
            pl.BlockSpec((RB, D), row),
            pl.BlockSpec((RB, D), lambda i: (i + nq, 0)),
            pl.BlockSpec((RB, 1), row),
            pl.BlockSpec((RB, 1), row),
        ],
        out_specs=pl.BlockSpec((RB, D), row),
        out_shape=jax.ShapeDtypeStruct((S, D), jnp.float32),
    )(h1, x2, sgT, suT, sdT, gathered, gathered, w0, w1)


def _sc_route(i0, i1):
    mesh = plsc.VectorSubcoreMesh(core_axis_name="c", subcore_axis_name="s")
    L = 16
    n_tok_ch = S // L

    @functools.partial(
        pl.kernel, mesh=mesh,
        out_type=(
            jax.ShapeDtypeStruct((2 * S,), jnp.int32),
            jax.ShapeDtypeStruct((32,), jnp.int32),
            jax.ShapeDtypeStruct((16,), jnp.int32),
        ),
        scratch_types=[
            pltpu.VMEM((S,), jnp.int32),
            pltpu.VMEM((S,), jnp.int32),
            pltpu.VMEM((2 * S,), jnp.int32),
            pltpu.VMEM((32,), jnp.int32),
            pltpu.VMEM((16,), jnp.int32),
        ],
    )
    def k(i0_hbm, i1_hbm, back_hbm, be_hbm, nl_hbm,
          i0_v, i1_v, back_v, be_v, nl_v):
        wid = lax.axis_index("s") * 2 + lax.axis_index("c")

        @pl.when(wid == 0)
        def _():
            pltpu.sync_copy(i0_hbm, i0_v)
            pltpu.sync_copy(i1_hbm, i1_v)
            lane = lax.iota(jnp.int32, L)
            zero = jnp.zeros((L,), jnp.int32)
            last = zero + (L - 1)
            dn = lax.GatherDimensionNumbers(offset_dims=(),
                                            collapsed_slice_dims=(0,),
                                            start_index_map=(0,))

            def perm(v, idx):
                return lax.gather(v, idx[:, None], dimension_numbers=dn,
                                  slice_sizes=(1,),
                                  mode=lax.GatherScatterMode.PROMISE_IN_BOUNDS)

            def incl_scan(v):
                for sh in (1, 2, 4, 8):
                    v = v + jnp.where(lane >= sh,
                                      perm(v, jnp.maximum(lane - sh, 0)), 0)
                return v

            def splat_last(v):
                return perm(v, last)

            def count_body(c, accs):
                e0 = i0_v[pl.ds(c * L, L)]
                e1 = i1_v[pl.ds(c * L, L)]
                return tuple(
                    accs[e]
                    + jnp.where(e0 == e, 1, 0)
                    + jnp.where(e1 == e, 1, 0)
                    for e in range(E))

            accs = lax.fori_loop(0, n_tok_ch, count_body, (zero,) * E)
            cnts = [splat_last(incl_scan(a)) for a in accs]
            pcs = [((c + BLK - 1) >> 8) << 8 for c in cnts]
            pss = [zero]
            for e in range(E):
                pss.append(pss[e] + pcs[e])
            nl_v[...] = pss[E] >> 8
            for c in range(2):
                bpos = (c * L + lane) * BLK
                acc = zero
                for e in range(E):
                    acc = acc + jnp.where(bpos >= pss[e] + pcs[e], 1, 0)
                be_v[pl.ds(c * L, L)] = jnp.minimum(acc, E - 1)

            def place_body(c, offs):
                for kk, ref in ((0, i0_v), (1, i1_v)):
                    e_vec = ref[pl.ds(c * L, L)]
                    dst = zero
                    new_offs = []
                    for e in range(E):
                        mi = jnp.where(e_vec == e, 1, 0)
                        inc = incl_scan(mi)
                        dst = jnp.where(e_vec == e, offs[e] + inc - mi, dst)
                        new_offs.append(offs[e] + splat_last(inc))
                    offs = tuple(new_offs)
                    back_v[pl.ds(kk * S + c * L, L)] = dst
                return offs

            lax.fori_loop(0, n_tok_ch, place_body, tuple(pss[:E]))

            pltpu.sync_copy(back_v, back_hbm)
            pltpu.sync_copy(be_v, be_hbm)
            pltpu.sync_copy(nl_v, nl_hbm)

    return k(i0, i1)


def _sc_scatter_rows(table, idx, out_rows):
    info = plsc.get_sparse_core_info()
    nw = info.num_cores * info.num_subcores
    n_tok, d = table.shape
    per_w = n_tok // nw
    mesh = plsc.VectorSubcoreMesh(core_axis_name="c", subcore_axis_name="s")

    @functools.partial(
        pl.kernel, mesh=mesh,
        out_type=jax.ShapeDtypeStruct((out_rows, d), table.dtype),
        scratch_types=[
            pltpu.VMEM((per_w,), jnp.int32),
            pltpu.VMEM((per_w,), jnp.int32),
            pltpu.VMEM((per_w, d), table.dtype),
            pltpu.SemaphoreType.DMA,
            pltpu.SemaphoreType.DMA,
        ],
    )
    def k(table_hbm, idx_hbm, out_hbm, idx0_v, idx1_v, buf, sem0, sem1):
        wid = lax.axis_index("s") * info.num_cores + lax.axis_index("c")
        tbase = wid * per_w
        pltpu.sync_copy(idx_hbm.at[pl.ds(tbase, per_w)], idx0_v)
        pltpu.sync_copy(idx_hbm.at[pl.ds(S + tbase, per_w)], idx1_v)
        pltpu.sync_copy(table_hbm.at[pl.ds(tbase, per_w)], buf)
        h0 = pltpu.async_copy(buf, out_hbm.at[idx0_v], sem0)
        h1 = pltpu.async_copy(buf, out_hbm.at[idx1_v], sem1)
        h0.wait()
        h1.wait()

    return k(table, idx)


def _sc_gather_rows(table, idx, chunk=32, nbuf=3):
    info = plsc.get_sparse_core_info()
    nw = info.num_cores * info.num_subcores
    n, tail = idx.shape[0], table.shape[1:]
    per_w = n // nw
    n_ch = per_w // chunk
    assert n_ch * chunk == per_w
    mesh = plsc.VectorSubcoreMesh(core_axis_name="c", subcore_axis_name="s")

    @functools.partial(
        pl.kernel, mesh=mesh,
        out_type=jax.ShapeDtypeStruct((n,) + tail, table.dtype),
        scratch_types=(
            [pltpu.VMEM((per_w,), jnp.int32)]
            + [pltpu.VMEM((chunk,) + tail, table.dtype)] * nbuf
            + [pltpu.SemaphoreType.DMA] * (2 * nbuf)
        ),
    )
    def k(table_hbm, idx_hbm, out_hbm, idx_v, *bufs_sems):
        bufs = bufs_sems[:nbuf]
        gsems = bufs_sems[nbuf:2 * nbuf]
        wsems = bufs_sems[2 * nbuf:]
        wid = lax.axis_index("s") * info.num_cores + lax.axis_index("c")
        base = wid * per_w
        pltpu.sync_copy(idx_hbm.at[pl.ds(base, per_w)], idx_v)

        def start_gather(c, b):
            return pltpu.async_copy(
                table_hbm.at[idx_v.at[pl.ds(c * chunk, chunk)]],
                bufs[b], gsems[b])

        gh, wh = {}, {}
        for c in range(min(nbuf, n_ch)):
            gh[c] = start_gather(c, c % nbuf)
        for c in range(n_ch):
            b = c % nbuf
            gh[c].wait()
            wh[c] = pltpu.async_copy(
                bufs[b], out_hbm.at[pl.ds(base + c * chunk, chunk)], wsems[b])
            if c + nbuf < n_ch:
                wh[c].wait()
                gh[c + nbuf] = start_gather(c + nbuf, b)
        for c in range(max(0, n_ch - nbuf), n_ch):
            wh[c].wait()

    return k(table, idx)


def kernel(hidden_states, position_ids, ln1_w, q_w, k_w, v_w, o_w, ln2_w,
           gate_w, eg, eu, ed, sg, su, sd):
    x = hidden_states.reshape(S, D)

    inv_freq = 1.0 / (ROPE_BASE ** (jnp.arange(0, HD, 2, dtype=jnp.float32) / HD))
    freqs = jnp.outer(jnp.arange(S, dtype=jnp.float32), inv_freq)
    emb = jnp.concatenate([freqs, freqs], axis=-1)
    pos = position_ids.reshape(S)
    cosE = jnp.tile(jnp.cos(emb)[pos], (1, H))
    sinE = jnp.tile(jnp.sin(emb)[pos], (1, H))

    wqT = q_w.T.astype(jnp.bfloat16)
    wkT = k_w.T.astype(jnp.bfloat16)
    wvT = v_w.T.astype(jnp.bfloat16)
    q, k, v = _qkv_rope(x, cosE, sinE, ln1_w.reshape(1, D), wqT, wkT, wvT)

    qh = q.reshape(S, H, HD).transpose(1, 0, 2)
    kh = k.reshape(S, H, HD).transpose(1, 0, 2)
    vh = v.reshape(S, H, HD).transpose(1, 0, 2)
    ao = _attention(qh, kh, vh).transpose(1, 0, 2).reshape(S, D)

    h1, x2, i0, i1, w0, w1 = _oproj_ln2_gate(x, ao,
                                             o_w.T.astype(jnp.bfloat16),
                                             ln2_w.reshape(1, D), gate_w.T)

    back_idx, be, nlive = _sc_route(i0.reshape(S), i1.reshape(S))
    xg = _sc_scatter_rows(x2, back_idx, NP_PAD)
    yg = _grouped_ffn(xg, eg, eu, ed, be, nlive)
    gathered = _sc_gather_rows(yg, back_idx)

    out = _shared_ffn(h1, x2, sg.T.astype(jnp.bfloat16),
                      su.T.astype(jnp.bfloat16), sd.T.astype(jnp.bfloat16),
                      gathered, w0, w1)
    return out.reshape(1, S, D)


# speedup vs baseline: 1.5014x; 1.0216x over previous
import functools

import jax
import jax.numpy as jnp
import numpy as np
from jax import lax
from jax.experimental import pallas as pl
from jax.experimental.pallas import tpu as pltpu
from jax.experimental.pallas import tpu_sc as plsc

S = 2048
D = 1024
H = 16
HD = 64
E = 8
DFF = 1408
SFF = 2816
EPS = 1e-6
ROPE_BASE = 10000.0

RB = 256
BLK = 256
NP_PAD = 4096 + 8 * (BLK - 1)
NP_PAD = ((NP_PAD + BLK - 1) // BLK) * BLK
NB = NP_PAD // BLK

def _rotate_half_flat(x):
    first = (lax.broadcasted_iota(jnp.int32, x.shape, 1) & 63) < 32
    return jnp.where(first, -jnp.roll(x, -32, axis=1), jnp.roll(x, 32, axis=1))


def _ntdot(a, b):
    return lax.dot_general(a, b, (((1,), (1,)), ((), ())),
                           preferred_element_type=jnp.float32)


def _k1_body(x_ref, cos_ref, sin_ref, ln1_ref, wq_ref, wk_ref,
             wv_ref, q_ref, k_ref, v_ref, wqs, wks, wvs):
    @pl.when(pl.program_id(0) == 0)
    def _():
        wqs[...] = wq_ref[...].astype(jnp.bfloat16)
        wks[...] = wk_ref[...].astype(jnp.bfloat16)
        wvs[...] = wv_ref[...].astype(jnp.bfloat16)

    x = x_ref[...]
    var = jnp.mean(x * x, axis=-1, keepdims=True)
    xn = ((x * lax.rsqrt(var + EPS)) * ln1_ref[...]).astype(jnp.bfloat16)
    c, s = cos_ref[...], sin_ref[...]
    q = _ntdot(xn, wqs[...])
    q_ref[...] = (q * c + _rotate_half_flat(q) * s).astype(jnp.bfloat16)
    k = _ntdot(xn, wks[...])
    k_ref[...] = (k * c + _rotate_half_flat(k) * s).astype(jnp.bfloat16)
    v_ref[...] = _ntdot(xn, wvs[...]).astype(jnp.bfloat16)


def _qkv_rope(x, cosE, sinE, ln1_w, wqT, wkT, wvT):
    row = lambda i: (i, 0)
    full = lambda i: (0, 0)
    return pl.pallas_call(
        _k1_body,
        grid=(S // RB,),
        in_specs=[
            pl.BlockSpec((RB, D), row),
            pl.BlockSpec((RB, D), row),
            pl.BlockSpec((RB, D), row),
            pl.BlockSpec((1, D), full),
            pl.BlockSpec((D, D), full),
            pl.BlockSpec((D, D), full),
            pl.BlockSpec((D, D), full),
        ],
        out_specs=[pl.BlockSpec((RB, D), row)] * 3,
        out_shape=[jax.ShapeDtypeStruct((S, D), jnp.bfloat16)] * 3,
        scratch_shapes=[pltpu.VMEM((D, D), jnp.bfloat16)] * 3,
    )(x, cosE, sinE, ln1_w, wqT, wkT, wvT)


def _attn_body(kv_len, qb0, q_ref, k_ref, v_ref, o_ref):
    q = q_ref[0]
    k = k_ref[0]
    s = lax.dot_general(q, k, (((1,), (1,)), ((), ())),
                        preferred_element_type=jnp.float32) * (1.0 / 8.0)
    qb = pl.program_id(1) + qb0
    rows = qb * RB + lax.broadcasted_iota(jnp.int32, (RB, kv_len), 0)
    cols = lax.broadcasted_iota(jnp.int32, (RB, kv_len), 1)
    s = jnp.where(rows >= cols, s, -1e30)
    m = jnp.max(s, axis=-1, keepdims=True)
    p = jnp.exp(s - m)
    p = (p / jnp.sum(p, axis=-1, keepdims=True)).astype(jnp.bfloat16)
    o_ref[0] = lax.dot_general(p, v_ref[0], (((1,), (0,)), ((), ())),
                               preferred_element_type=jnp.float32
                               ).astype(jnp.bfloat16)


def _attention_part(qh, kh, vh, qb0, n_qb, kv_len):
    return pl.pallas_call(
        functools.partial(_attn_body, kv_len, qb0),
        grid=(H, n_qb),
        in_specs=[
            pl.BlockSpec((1, RB, HD), lambda h, qb: (h, qb + qb0, 0)),
            pl.BlockSpec((1, kv_len, HD), lambda h, qb: (h, 0, 0)),
            pl.BlockSpec((1, kv_len, HD), lambda h, qb: (h, 0, 0)),
        ],
        out_specs=pl.BlockSpec((1, RB, HD), lambda h, qb: (h, qb, 0)),
        out_shape=jax.ShapeDtypeStruct((H, n_qb * RB, HD), jnp.bfloat16),
    )(qh, kh, vh)


def _attention(qh, kh, vh):
    nq = S // RB
    step = nq // 4
    parts = [
        _attention_part(qh, kh, vh, i * step, step, (i + 1) * step * RB)
        for i in range(4)
    ]
    return jnp.concatenate(parts, axis=1)


def _k3_body(x_ref, ao_ref, ow_ref, ln2_ref, gw_ref, h1_ref, x2_ref,
             i0_ref, i1_ref, w0_ref, w1_ref, ows):
    @pl.when(pl.program_id(0) == 0)
    def _():
        ows[...] = ow_ref[...].astype(jnp.bfloat16)

    proj = _ntdot(ao_ref[...], ows[...])
    h1 = x_ref[...] + proj
    h1_ref[...] = h1
    var = jnp.mean(h1 * h1, axis=-1, keepdims=True)
    x2 = (h1 * lax.rsqrt(var + EPS)) * ln2_ref[...]
    x2_ref[...] = x2
    lg = _ntdot(x2, gw_ref[...])
    eidx = lax.broadcasted_iota(jnp.int32, (RB, E), 1)
    m1 = jnp.max(lg, axis=-1, keepdims=True)
    i0 = jnp.min(jnp.where(lg == m1, eidx, E), axis=-1, keepdims=True)
    lg2 = jnp.where(eidx == i0, -jnp.inf, lg)
    m2 = jnp.max(lg2, axis=-1, keepdims=True)
    i1 = jnp.min(jnp.where(lg2 == m2, eidx, E), axis=-1, keepdims=True)
    z = jnp.sum(jnp.exp(lg - m1), axis=-1, keepdims=True)
    i0_ref[...] = i0
    i1_ref[...] = i1
    w0_ref[...] = 1.0 / z
    w1_ref[...] = jnp.exp(m2 - m1) / z


def _oproj_ln2_gate(x, ao, owT, ln2_w, gwT):
    row = lambda i: (i, 0)
    full = lambda i: (0, 0)
    return pl.pallas_call(
        _k3_body,
        grid=(S // RB,),
        in_specs=[
            pl.BlockSpec((RB, D), row),
            pl.BlockSpec((RB, D), row),
            pl.BlockSpec((D, D), full),
            pl.BlockSpec((1, D), full),
            pl.BlockSpec((E, D), full),
        ],
        out_specs=[
            pl.BlockSpec((RB, D), row),
            pl.BlockSpec((RB, D), row),
            pl.BlockSpec((RB, 1), row),
            pl.BlockSpec((RB, 1), row),
            pl.BlockSpec((RB, 1), row),
            pl.BlockSpec((RB, 1), row),
        ],
        out_shape=[
            jax.ShapeDtypeStruct((S, D), jnp.float32),
            jax.ShapeDtypeStruct((S, D), jnp.float32),
            jax.ShapeDtypeStruct((S, 1), jnp.int32),
            jax.ShapeDtypeStruct((S, 1), jnp.int32),
            jax.ShapeDtypeStruct((S, 1), jnp.float32),
            jax.ShapeDtypeStruct((S, 1), jnp.float32),
        ],
        scratch_shapes=[pltpu.VMEM((D, D), jnp.bfloat16)],
    )(x, ao, owT, ln2_w, gwT)


def _silu(a):
    return a * (1.0 / (1.0 + jnp.exp(-a)))


def _moe_body(nlive_ref, be_ref, xg_ref, eg_ref, eu_ref, ed_ref, yg_ref,
              egb_ref, eub_ref, edb_ref):
    b = pl.program_id(0)

    @pl.when(b < nlive_ref[0])
    def _():
        prev = be_ref[jnp.maximum(b - 1, 0)]
        changed = jnp.logical_or(b == 0, be_ref[b] != prev)

        @pl.when(changed)
        def _():
            egb_ref[...] = eg_ref[0].astype(jnp.bfloat16)
            eub_ref[...] = eu_ref[0].astype(jnp.bfloat16)
            edb_ref[...] = ed_ref[0].astype(jnp.bfloat16)

        xb = xg_ref[...].astype(jnp.bfloat16)
        a = lax.dot_general(xb, egb_ref[...], (((1,), (1,)), ((), ())),
                            preferred_element_type=jnp.float32)
        u = lax.dot_general(xb, eub_ref[...], (((1,), (1,)), ((), ())),
                            preferred_element_type=jnp.float32)
        s = (_silu(a) * u).astype(jnp.bfloat16)
        yg_ref[...] = lax.dot_general(s, edb_ref[...], (((1,), (1,)), ((), ())),
                                      preferred_element_type=jnp.float32)


def _grouped_ffn(xg, egb, eub, edb, be, nlive):
    grid_spec = pltpu.PrefetchScalarGridSpec(
        num_scalar_prefetch=2,
        grid=(NB,),
        in_specs=[
            pl.BlockSpec((BLK, D), lambda b, nl, be: (b, 0)),
            pl.BlockSpec((1, DFF, D), lambda b, nl, be: (be[b], 0, 0)),
            pl.BlockSpec((1, DFF, D), lambda b, nl, be: (be[b], 0, 0)),
            pl.BlockSpec((1, D, DFF), lambda b, nl, be: (be[b], 0, 0)),
        ],
        out_specs=pl.BlockSpec((BLK, D), lambda b, nl, be: (b, 0)),
        scratch_shapes=[
            pltpu.VMEM((DFF, D), jnp.bfloat16),
            pltpu.VMEM((DFF, D), jnp.bfloat16),
            pltpu.VMEM((D, DFF), jnp.bfloat16),
        ],
    )
    return pl.pallas_call(
        _moe_body,
        grid_spec=grid_spec,
        out_shape=jax.ShapeDtypeStruct((NP_PAD, D), jnp.float32),
    )(nlive, be, xg, egb, eub, edb)


def _shared_body(h1_ref, x2_ref, sg_ref, su_ref, sd_ref, g0_ref, g1_ref,
                 w0_ref, w1_ref, o_ref):
    xb = x2_ref[...].astype(jnp.bfloat16)
    a = jnp.dot(xb, sg_ref[...], preferred_element_type=jnp.float32)
    u = jnp.dot(xb, su_ref[...], preferred_element_type=jnp.float32)
    s = (_silu(a) * u).astype(jnp.bfloat16)
    o_ref[...] = (h1_ref[...]
                  + jnp.dot(s, sd_ref[...], preferred_element_type=jnp.float32)
                  + w0_ref[...] * g0_ref[...] + w1_ref[...] * g1_ref[...])


def _shared_ffn(h1, x2, sgT, suT, sdT, gathered, w0, w1):
    row = lambda i: (i, 0)
    full = lambda i: (0, 0)
    nq = S // RB
    return pl.pallas_call(
        _shared_body,
        grid=(nq,),
        in_specs=[
            pl.BlockSpec((RB, D), row),
            pl.BlockSpec((RB, D), row),
            pl.BlockSpec((D, SFF), full),
            pl.BlockSpec((D, SFF), full),
            pl.BlockSpec((SFF, D), full),
            pl.BlockSpec((RB, D), row),
            pl.BlockSpec((RB, D), lambda i: (i + nq, 0)),
            pl.BlockSpec((RB, 1), row),
            pl.BlockSpec((RB, 1), row),
        ],
        out_specs=pl.BlockSpec((RB, D), row),
        out_shape=jax.ShapeDtypeStruct((S, D), jnp.float32),
    )(h1, x2, sgT, suT, sdT, gathered, gathered, w0, w1)


def _sc_route(i0, i1):
    mesh = plsc.VectorSubcoreMesh(core_axis_name="c", subcore_axis_name="s")
    L = 16
    n_tok_ch = S // L

    @functools.partial(
        pl.kernel, mesh=mesh,
        out_type=(
            jax.ShapeDtypeStruct((2 * S,), jnp.int32),
            jax.ShapeDtypeStruct((32,), jnp.int32),
            jax.ShapeDtypeStruct((16,), jnp.int32),
        ),
        scratch_types=[
            pltpu.VMEM((S,), jnp.int32),
            pltpu.VMEM((S,), jnp.int32),
            pltpu.VMEM((2 * S,), jnp.int32),
            pltpu.VMEM((32,), jnp.int32),
            pltpu.VMEM((16,), jnp.int32),
        ],
    )
    def k(i0_hbm, i1_hbm, back_hbm, be_hbm, nl_hbm,
          i0_v, i1_v, back_v, be_v, nl_v):
        wid = lax.axis_index("s") * 2 + lax.axis_index("c")

        @pl.when(wid == 0)
        def _():
            pltpu.sync_copy(i0_hbm, i0_v)
            pltpu.sync_copy(i1_hbm, i1_v)
            lane = lax.iota(jnp.int32, L)
            zero = jnp.zeros((L,), jnp.int32)
            last = zero + (L - 1)
            dn = lax.GatherDimensionNumbers(offset_dims=(),
                                            collapsed_slice_dims=(0,),
                                            start_index_map=(0,))

            def perm(v, idx):
                return lax.gather(v, idx[:, None], dimension_numbers=dn,
                                  slice_sizes=(1,),
                                  mode=lax.GatherScatterMode.PROMISE_IN_BOUNDS)

            def incl_scan(v):
                for sh in (1, 2, 4, 8):
                    v = v + jnp.where(lane >= sh,
                                      perm(v, jnp.maximum(lane - sh, 0)), 0)
                return v

            def splat_last(v):
                return perm(v, last)

            def count_body(c, accs):
                e0 = i0_v[pl.ds(c * L, L)]
                e1 = i1_v[pl.ds(c * L, L)]
                return tuple(
                    accs[e]
                    + jnp.where(e0 == e, 1, 0)
                    + jnp.where(e1 == e, 1, 0)
                    for e in range(E))

            accs = lax.fori_loop(0, n_tok_ch, count_body, (zero,) * E)
            cnts = [splat_last(incl_scan(a)) for a in accs]
            pcs = [((c + BLK - 1) >> 8) << 8 for c in cnts]
            pss = [zero]
            for e in range(E):
                pss.append(pss[e] + pcs[e])
            nl_v[...] = pss[E] >> 8
            for c in range(2):
                bpos = (c * L + lane) * BLK
                acc = zero
                for e in range(E):
                    acc = acc + jnp.where(bpos >= pss[e] + pcs[e], 1, 0)
                be_v[pl.ds(c * L, L)] = jnp.minimum(acc, E - 1)

            def place_body(c, offs):
                for kk, ref in ((0, i0_v), (1, i1_v)):
                    e_vec = ref[pl.ds(c * L, L)]
                    dst = zero
                    new_offs = []
                    for e in range(E):
                        mi = jnp.where(e_vec == e, 1, 0)
                        inc = incl_scan(mi)
                        dst = jnp.where(e_vec == e, offs[e] + inc - mi, dst)
                        new_offs.append(offs[e] + splat_last(inc))
                    offs = tuple(new_offs)
                    back_v[pl.ds(kk * S + c * L, L)] = dst
                return offs

            lax.fori_loop(0, n_tok_ch, place_body, tuple(pss[:E]))

            pltpu.sync_copy(back_v, back_hbm)
            pltpu.sync_copy(be_v, be_hbm)
            pltpu.sync_copy(nl_v, nl_hbm)

    return k(i0, i1)


def _sc_scatter_rows(table, idx, out_rows):
    info = plsc.get_sparse_core_info()
    nw = info.num_cores * info.num_subcores
    n_tok, d = table.shape
    per_w = n_tok // nw
    mesh = plsc.VectorSubcoreMesh(core_axis_name="c", subcore_axis_name="s")

    @functools.partial(
        pl.kernel, mesh=mesh,
        out_type=jax.ShapeDtypeStruct((out_rows, d), table.dtype),
        scratch_types=[
            pltpu.VMEM((per_w,), jnp.int32),
            pltpu.VMEM((per_w,), jnp.int32),
            pltpu.VMEM((per_w, d), table.dtype),
            pltpu.SemaphoreType.DMA,
            pltpu.SemaphoreType.DMA,
        ],
    )
    def k(table_hbm, idx_hbm, out_hbm, idx0_v, idx1_v, buf, sem0, sem1):
        wid = lax.axis_index("s") * info.num_cores + lax.axis_index("c")
        tbase = wid * per_w
        pltpu.sync_copy(idx_hbm.at[pl.ds(tbase, per_w)], idx0_v)
        pltpu.sync_copy(idx_hbm.at[pl.ds(S + tbase, per_w)], idx1_v)
        pltpu.sync_copy(table_hbm.at[pl.ds(tbase, per_w)], buf)
        h0 = pltpu.async_copy(buf, out_hbm.at[idx0_v], sem0)
        h1 = pltpu.async_copy(buf, out_hbm.at[idx1_v], sem1)
        h0.wait()
        h1.wait()

    return k(table, idx)


def _sc_gather_rows(table, idx, chunk=32, nbuf=3):
    info = plsc.get_sparse_core_info()
    nw = info.num_cores * info.num_subcores
    n, tail = idx.shape[0], table.shape[1:]
    per_w = n // nw
    n_ch = per_w // chunk
    assert n_ch * chunk == per_w
    mesh = plsc.VectorSubcoreMesh(core_axis_name="c", subcore_axis_name="s")

    @functools.partial(
        pl.kernel, mesh=mesh,
        out_type=jax.ShapeDtypeStruct((n,) + tail, table.dtype),
        scratch_types=(
            [pltpu.VMEM((per_w,), jnp.int32)]
            + [pltpu.VMEM((chunk,) + tail, table.dtype)] * nbuf
            + [pltpu.SemaphoreType.DMA] * (2 * nbuf)
        ),
    )
    def k(table_hbm, idx_hbm, out_hbm, idx_v, *bufs_sems):
        bufs = bufs_sems[:nbuf]
        gsems = bufs_sems[nbuf:2 * nbuf]
        wsems = bufs_sems[2 * nbuf:]
        wid = lax.axis_index("s") * info.num_cores + lax.axis_index("c")
        base = wid * per_w
        pltpu.sync_copy(idx_hbm.at[pl.ds(base, per_w)], idx_v)

        def start_gather(c, b):
            return pltpu.async_copy(
                table_hbm.at[idx_v.at[pl.ds(c * chunk, chunk)]],
                bufs[b], gsems[b])

        gh, wh = {}, {}
        for c in range(min(nbuf, n_ch)):
            gh[c] = start_gather(c, c % nbuf)
        for c in range(n_ch):
            b = c % nbuf
            gh[c].wait()
            wh[c] = pltpu.async_copy(
                bufs[b], out_hbm.at[pl.ds(base + c * chunk, chunk)], wsems[b])
            if c + nbuf < n_ch:
                wh[c].wait()
                gh[c + nbuf] = start_gather(c + nbuf, b)
        for c in range(max(0, n_ch - nbuf), n_ch):
            wh[c].wait()

    return k(table, idx)


def kernel(hidden_states, position_ids, ln1_w, q_w, k_w, v_w, o_w, ln2_w,
           gate_w, eg, eu, ed, sg, su, sd):
    x = hidden_states.reshape(S, D)

    inv_freq = 1.0 / (ROPE_BASE ** (jnp.arange(0, HD, 2, dtype=jnp.float32) / HD))
    freqs = jnp.outer(jnp.arange(S, dtype=jnp.float32), inv_freq)
    emb = jnp.concatenate([freqs, freqs], axis=-1)
    pos = position_ids.reshape(S)
    cosE = jnp.tile(jnp.cos(emb)[pos], (1, H))
    sinE = jnp.tile(jnp.sin(emb)[pos], (1, H))

    wqT = q_w.T.astype(jnp.bfloat16)
    wkT = k_w.T.astype(jnp.bfloat16)
    wvT = v_w.T.astype(jnp.bfloat16)
    q, k, v = _qkv_rope(x, cosE, sinE, ln1_w.reshape(1, D), q_w, k_w, v_w)

    qh = q.reshape(S, H, HD).transpose(1, 0, 2)
    kh = k.reshape(S, H, HD).transpose(1, 0, 2)
    vh = v.reshape(S, H, HD).transpose(1, 0, 2)
    ao = _attention(qh, kh, vh).transpose(1, 0, 2).reshape(S, D)

    h1, x2, i0, i1, w0, w1 = _oproj_ln2_gate(x, ao, o_w,
                                             ln2_w.reshape(1, D), gate_w)

    back_idx, be, nlive = _sc_route(i0.reshape(S), i1.reshape(S))
    xg = _sc_scatter_rows(x2, back_idx, NP_PAD)
    yg = _grouped_ffn(xg, eg, eu, ed, be, nlive)
    gathered = _sc_gather_rows(yg, back_idx)

    out = _shared_ffn(h1, x2, sg.T.astype(jnp.bfloat16),
                      su.T.astype(jnp.bfloat16), sd.T.astype(jnp.bfloat16),
                      gathered, w0, w1)
    return out.reshape(1, S, D)
